# speedup vs baseline: 1.8186x; 1.8186x over previous
import functools
import jax
import jax.numpy as jnp
from jax import lax
from jax.experimental import pallas as pl
from jax.experimental.pallas import tpu as pltpu

NUM_REL = 3
NUM_GRAPHS = 64
ROW_BLOCK = 2000


def _dense_body(h_ref, agg_ref, icnt_ref, wroot_ref, b_ref, w_ref, out_ref):
    h = h_ref[...]
    acc = jnp.dot(h, wroot_ref[...], preferred_element_type=jnp.float32)
    acc = acc + b_ref[...][None, :]
    for r in range(NUM_REL):
        m = agg_ref[r] * icnt_ref[:, r][:, None]
        acc = acc + jnp.dot(m, w_ref[r], preferred_element_type=jnp.float32)
    out_ref[...] = jnp.maximum(acc, 0.0)


def _tc_dense(h, agg, icnt, wroot, b, w):
    n, d = h.shape
    hh = w.shape[2]
    grid = n // ROW_BLOCK
    return pl.pallas_call(
        _dense_body,
        grid=(grid,),
        in_specs=[
            pl.BlockSpec((ROW_BLOCK, d), lambda i: (i, 0)),
            pl.BlockSpec((NUM_REL, ROW_BLOCK, d), lambda i: (0, i, 0)),
            pl.BlockSpec((ROW_BLOCK, NUM_REL), lambda i: (i, 0)),
            pl.BlockSpec((d, hh), lambda i: (0, 0)),
            pl.BlockSpec((hh,), lambda i: (0,)),
            pl.BlockSpec((NUM_REL, d, hh), lambda i: (0, 0, 0)),
        ],
        out_specs=pl.BlockSpec((ROW_BLOCK, hh), lambda i: (i, 0)),
        out_shape=jax.ShapeDtypeStruct((n, hh), jnp.float32),
    )(h, agg, icnt, wroot, b, w)


def _pool_body(h_ref, batch_ref, linw_ref, linb_ref, out_ref, acc_ref, cnt_ref):
    i = pl.program_id(0)

    @pl.when(i == 0)
    def _init():
        acc_ref[...] = jnp.zeros_like(acc_ref)
        cnt_ref[...] = jnp.zeros_like(cnt_ref)

    bvec = batch_ref[0, 0, :]
    iota = lax.broadcasted_iota(jnp.int32, (ROW_BLOCK, NUM_GRAPHS), 1)
    onehot = (bvec[:, None] == iota).astype(jnp.float32)
    acc_ref[...] += lax.dot_general(
        onehot, h_ref[...], (((0,), (0,)), ((), ())),
        preferred_element_type=jnp.float32)
    cnt_ref[...] += jnp.sum(onehot, axis=0, keepdims=True)

    @pl.when(i == pl.num_programs(0) - 1)
    def _fin():
        pooled = acc_ref[...] / jnp.maximum(cnt_ref[...], 1.0).T
        out_ref[...] = jnp.dot(pooled, linw_ref[...],
                               preferred_element_type=jnp.float32) + linb_ref[...][None, :]


def _tc_pool(h, batch3d, linw, linb):
    n, d = h.shape
    c = linw.shape[1]
    grid = n // ROW_BLOCK
    return pl.pallas_call(
        _pool_body,
        grid=(grid,),
        in_specs=[
            pl.BlockSpec((ROW_BLOCK, d), lambda i: (i, 0)),
            pl.BlockSpec((1, 1, ROW_BLOCK), lambda i: (i, 0, 0)),
            pl.BlockSpec((d, c), lambda i: (0, 0)),
            pl.BlockSpec((c,), lambda i: (0,)),
        ],
        out_specs=pl.BlockSpec((NUM_GRAPHS, c), lambda i: (0, 0)),
        out_shape=jax.ShapeDtypeStruct((NUM_GRAPHS, c), jnp.float32),
        scratch_shapes=[
            pltpu.VMEM((NUM_GRAPHS, d), jnp.float32),
            pltpu.VMEM((1, NUM_GRAPHS), jnp.float32),
        ],
    )(h, batch3d, linw, linb)


def _aggregate(h_rows, comb, n):
    agg = jax.ops.segment_sum(h_rows, comb, num_segments=NUM_REL * n)
    return agg.reshape(NUM_REL, n, h_rows.shape[1])


def kernel(x, edge_index, edge_type, batch, embed, W1, Wroot1, b1, W2, Wroot2, b2, linW, linb):
    n = x.shape[0]
    src, dst = edge_index[0], edge_index[1]
    comb = edge_type * n + dst

    h0 = embed[x]
    cnt = jax.ops.segment_sum(jnp.ones(src.shape, jnp.float32), comb,
                              num_segments=NUM_REL * n).reshape(NUM_REL, n)
    icnt = (1.0 / jnp.maximum(cnt, 1.0)).T
    batch3d = batch.reshape(n // ROW_BLOCK, 1, ROW_BLOCK)

    agg1 = _aggregate(h0[src], comb, n)
    h1 = _tc_dense(h0, agg1, icnt, Wroot1, b1, W1)
    agg2 = _aggregate(h1[src], comb, n)
    h2 = _tc_dense(h1, agg2, icnt, Wroot2, b2, W2)
    return _tc_pool(h2, batch3d, linW, linb)


# speedup vs baseline: 10.4248x; 5.7323x over previous
import functools
import jax
import jax.numpy as jnp
from jax import lax
from jax.experimental import pallas as pl
from jax.experimental.pallas import tpu as pltpu
from jax.experimental.pallas import tpu_sc as plsc

NUM_REL = 3
NUM_GRAPHS = 64
ROW_BLOCK = 2000

NSC = 2
NTILE = 16
NW = NSC * NTILE


def _mesh():
    return plsc.VectorSubcoreMesh(core_axis_name="c", subcore_axis_name="s")


def _wid():
    return lax.axis_index("s") * NSC + lax.axis_index("c")


def _sc_embed_gather(x_pad, embed):
    npad, d = x_pad.shape[0], embed.shape[1]
    nch = npad // 128
    per_w = nch // NW

    def body(x_hbm, table_hbm, out_hbm, idx_v, rows_v, sem):
        w = _wid()

        def step(j, _):
            k = w * per_w + j
            pltpu.sync_copy(x_hbm.at[pl.ds(k * 128, 128)], idx_v)
            pltpu.async_copy(table_hbm.at[idx_v], rows_v, sem).wait()
            pltpu.sync_copy(rows_v, out_hbm.at[pl.ds(k * 128, 128), :])
            return ()

        lax.fori_loop(0, per_w, step, ())

    f = pl.kernel(
        body,
        out_type=jax.ShapeDtypeStruct((npad, d), jnp.float32),
        mesh=_mesh(),
        scratch_types=[
            pltpu.VMEM((128,), jnp.int32),
            pltpu.VMEM((128, d), jnp.float32),
            pltpu.SemaphoreType.DMA,
        ],
        compiler_params=pltpu.CompilerParams(use_tc_tiling_on_sc=False, needs_layout_passes=False),
    )
    return f(x_pad, embed)


def _dense_body(h_ref, agg_ref, icnt_ref, wroot_ref, b_ref, w_ref, out_ref):
    h = h_ref[...]
    acc = jnp.dot(h, wroot_ref[...], preferred_element_type=jnp.float32)
    acc = acc + b_ref[...][None, :]
    for r in range(NUM_REL):
        m = agg_ref[r] * icnt_ref[:, r][:, None]
        acc = acc + jnp.dot(m, w_ref[r], preferred_element_type=jnp.float32)
    out_ref[...] = jnp.maximum(acc, 0.0)


def _tc_dense(h, agg, icnt, wroot, b, w, n=None):
    d = h.shape[1]
    if n is None:
        n = h.shape[0]
    hh = w.shape[2]
    grid = n // ROW_BLOCK
    return pl.pallas_call(
        _dense_body,
        grid=(grid,),
        in_specs=[
            pl.BlockSpec((ROW_BLOCK, d), lambda i: (i, 0)),
            pl.BlockSpec((NUM_REL, ROW_BLOCK, d), lambda i: (0, i, 0)),
            pl.BlockSpec((ROW_BLOCK, NUM_REL), lambda i: (i, 0)),
            pl.BlockSpec((d, hh), lambda i: (0, 0)),
            pl.BlockSpec((hh,), lambda i: (0,)),
            pl.BlockSpec((NUM_REL, d, hh), lambda i: (0, 0, 0)),
        ],
        out_specs=pl.BlockSpec((ROW_BLOCK, hh), lambda i: (i, 0)),
        out_shape=jax.ShapeDtypeStruct((n, hh), jnp.float32),
    )(h, agg, icnt, wroot, b, w)


def _pool_body(h_ref, batch_ref, linw_ref, linb_ref, out_ref, acc_ref, cnt_ref):
    i = pl.program_id(0)

    @pl.when(i == 0)
    def _init():
        acc_ref[...] = jnp.zeros_like(acc_ref)
        cnt_ref[...] = jnp.zeros_like(cnt_ref)

    bvec = batch_ref[0, 0, :]
    iota = lax.broadcasted_iota(jnp.int32, (ROW_BLOCK, NUM_GRAPHS), 1)
    onehot = (bvec[:, None] == iota).astype(jnp.float32)
    acc_ref[...] += lax.dot_general(
        onehot, h_ref[...], (((0,), (0,)), ((), ())),
        preferred_element_type=jnp.float32)
    cnt_ref[...] += jnp.sum(onehot, axis=0, keepdims=True)

    @pl.when(i == pl.num_programs(0) - 1)
    def _fin():
        pooled = acc_ref[...] / jnp.maximum(cnt_ref[...], 1.0).T
        out_ref[...] = jnp.dot(pooled, linw_ref[...],
                               preferred_element_type=jnp.float32) + linb_ref[...][None, :]


def _tc_pool(h, batch3d, linw, linb):
    n, d = h.shape
    c = linw.shape[1]
    grid = n // ROW_BLOCK
    return pl.pallas_call(
        _pool_body,
        grid=(grid,),
        in_specs=[
            pl.BlockSpec((ROW_BLOCK, d), lambda i: (i, 0)),
            pl.BlockSpec((1, 1, ROW_BLOCK), lambda i: (i, 0, 0)),
            pl.BlockSpec((d, c), lambda i: (0, 0)),
            pl.BlockSpec((c,), lambda i: (0,)),
        ],
        out_specs=pl.BlockSpec((NUM_GRAPHS, c), lambda i: (0, 0)),
        out_shape=jax.ShapeDtypeStruct((NUM_GRAPHS, c), jnp.float32),
        scratch_shapes=[
            pltpu.VMEM((NUM_GRAPHS, d), jnp.float32),
            pltpu.VMEM((1, NUM_GRAPHS), jnp.float32),
        ],
    )(h, batch3d, linw, linb)


NB = 8
NCK = 6250
PADC = 6400
DUM = NCK
ET = 25600
CHK = 1600
CAPROWS = 6400
SCP = NB // NSC


def _iota16():
    return lax.iota(jnp.int32, 16)


def _extract(vec16, i):
    return jnp.sum(jnp.where(_iota16() == i, vec16, 0))


def _sc_params():
    return pltpu.CompilerParams(use_tc_tiling_on_sc=False, needs_layout_passes=False)


def _sc_count(dst_pad):
    def body(dst_hbm, out_hbm, chunk_v, row_v, sem):
        w = _wid()

        def chunk_body(ci, cnts):
            pltpu.sync_copy(dst_hbm.at[pl.ds(w * ET + ci * CHK, CHK)], chunk_v)

            def vec_body(vi, cnts):
                d = chunk_v[pl.ds(vi * 16, 16)]
                bkt = d // NCK
                return tuple(cnts[b] + (bkt == b).astype(jnp.int32)
                             for b in range(NB))

            return lax.fori_loop(0, CHK // 16, vec_body, cnts)

        cnts = lax.fori_loop(0, ET // CHK, chunk_body,
                             tuple(jnp.zeros((16,), jnp.int32) for _ in range(NB)))
        row = jnp.zeros((16,), jnp.int32)
        for b in range(NB):
            row = jnp.where(_iota16() == b, jnp.sum(cnts[b]), row)
        row_v[...] = row
        pltpu.sync_copy(row_v, out_hbm.at[w])

    f = pl.kernel(
        body,
        out_type=jax.ShapeDtypeStruct((NW, 16), jnp.int32),
        mesh=_mesh(),
        scratch_types=[
            pltpu.VMEM((CHK,), jnp.int32),
            pltpu.VMEM((16,), jnp.int32),
            pltpu.SemaphoreType.DMA,
        ],
        compiler_params=_sc_params(),
    )
    return f(dst_pad)


def _sc_partition(src_pad, dst_pad, typ_pad, slot_off, loc_off, cnt_wb, dummy_meta):
    STG = ET + NB * 16

    def body(src_hbm, dst_hbm, typ_hbm, slot_hbm, loc_hbm, cnt_hbm, dmy_hbm,
             srcp_hbm, combp_hbm,
             srcv, dstv, typv, sstage, cstage, m16, dzero, ddum, sem):
        w = _wid()

        pltpu.sync_copy(slot_hbm.at[w], m16)
        slot = m16[...]
        pltpu.sync_copy(loc_hbm.at[w], m16)
        loc = m16[...]
        pltpu.sync_copy(cnt_hbm.at[w], m16)
        cnt = m16[...]

        for i in range(8):
            dzero[pl.ds(i * 16, 16)] = jnp.zeros((16,), jnp.int32)
            ddum[pl.ds(i * 16, 16)] = jnp.full((16,), DUM, jnp.int32)

        @pl.when(w < NB)
        def _dummies():
            pltpu.sync_copy(dmy_hbm.at[0], m16)
            offa = pl.multiple_of(_extract(m16[...], w), 8)
            pltpu.sync_copy(dmy_hbm.at[1], m16)
            offb = pl.multiple_of(_extract(m16[...], w), 8)
            pltpu.sync_copy(dzero, srcp_hbm.at[pl.ds(offa, 128)])
            pltpu.sync_copy(dzero, srcp_hbm.at[pl.ds(offb, 128)])
            pltpu.sync_copy(ddum, combp_hbm.at[pl.ds(offa, 128)])
            pltpu.sync_copy(ddum, combp_hbm.at[pl.ds(offb, 128)])

        init = tuple(_extract(loc, b) for b in range(NB))

        def chunk_body(ci, cur):
            base = w * ET + ci * CHK
            pltpu.sync_copy(src_hbm.at[pl.ds(base, CHK)], srcv)
            pltpu.sync_copy(dst_hbm.at[pl.ds(base, CHK)], dstv)
            pltpu.sync_copy(typ_hbm.at[pl.ds(base, CHK)], typv)

            def vec_body(vi, cur):
                s = srcv[pl.ds(vi * 16, 16)]
                d = dstv[pl.ds(vi * 16, 16)]
                t = typv[pl.ds(vi * 16, 16)]
                bkt = d // NCK
                cb = t * PADC + (d - bkt * NCK)
                out = []
                for b in range(NB):
                    m = bkt == b
                    mi = m.astype(jnp.int32)
                    off = cur[b] + plsc.cumsum(mi) - 1
                    plsc.store_scatter(sstage, [off], s, mask=m)
                    plsc.store_scatter(cstage, [off], cb, mask=m)
                    out.append(cur[b] + jnp.sum(mi))
                return tuple(out)

            return lax.fori_loop(0, CHK // 16, vec_body, cur)

        cur = lax.fori_loop(0, ET // CHK, chunk_body, init)

        for b in range(NB):
            nb_cnt = _extract(cnt, b)
            pc = (nb_cnt + 7) & ~jnp.int32(7)
            toff = cur[b] + _iota16()
            tm = _iota16() < (pc - nb_cnt)
            plsc.store_scatter(sstage, [toff], jnp.zeros((16,), jnp.int32),
                               mask=tm)
            plsc.store_scatter(cstage, [toff], jnp.full((16,), DUM, jnp.int32),
                               mask=tm)
            lo = _extract(loc, b)
            go = _extract(slot, b)

            def drain(step, j0):
                def cond(j):
                    return j + step <= pc

                def dbody(j):
                    lj = pl.multiple_of(lo + j, 8)
                    gj = pl.multiple_of(go + j, 8)
                    pltpu.sync_copy(sstage.at[pl.ds(lj, step)],
                                    srcp_hbm.at[pl.ds(gj, step)])
                    pltpu.sync_copy(cstage.at[pl.ds(lj, step)],
                                    combp_hbm.at[pl.ds(gj, step)])
                    return j + step

                return lax.while_loop(cond, dbody, j0)

            j = drain(512, jnp.int32(0))
            j = drain(64, j)
            drain(8, j)

    f = pl.kernel(
        body,
        out_type=(jax.ShapeDtypeStruct((CAPROWS * 128,), jnp.int32),
                  jax.ShapeDtypeStruct((CAPROWS * 128,), jnp.int32)),
        mesh=_mesh(),
        scratch_types=[
            pltpu.VMEM((CHK,), jnp.int32),
            pltpu.VMEM((CHK,), jnp.int32),
            pltpu.VMEM((CHK,), jnp.int32),
            pltpu.VMEM((STG,), jnp.int32),
            pltpu.VMEM((STG,), jnp.int32),
            pltpu.VMEM((16,), jnp.int32),
            pltpu.VMEM((128,), jnp.int32),
            pltpu.VMEM((128,), jnp.int32),
            pltpu.SemaphoreType.DMA,
        ],
        compiler_params=_sc_params(),
    )
    return f(src_pad, dst_pad, typ_pad, slot_off, loc_off, cnt_wb, dummy_meta)


def _sc_hist(comb_part, rowmeta):
    def body(comb_hbm, meta_hbm, out_hbm, m16, cidx, hist, sem):
        w = _wid()
        b = w // 4
        pltpu.sync_copy(meta_hbm.at[0], m16)
        meta = m16[...]
        start = _extract(meta, b)
        end = _extract(meta, b + 1)

        def zbody(i, _):
            hist[pl.ds(i * 16, 16)] = jnp.zeros((16,), jnp.float32)
            return ()

        lax.fori_loop(0, NUM_REL * PADC // 16, zbody, ())

        def cond(k):
            return k < end

        def kbody(k):
            pltpu.sync_copy(comb_hbm.at[pl.ds(k * 128, 128)], cidx)

            def vbody(vi, _):
                c = cidx[pl.ds(vi * 16, 16)]
                plsc.addupdate_scatter(hist, [c], jnp.ones((16,), jnp.float32))
                return ()

            lax.fori_loop(0, 8, vbody, ())
            return k + 4

        lax.while_loop(cond, kbody, start + (w % 4))
        pltpu.sync_copy(hist, out_hbm.at[w])

    f = pl.kernel(
        body,
        out_type=jax.ShapeDtypeStruct((NW, NUM_REL * PADC), jnp.float32),
        mesh=_mesh(),
        scratch_types=[
            pltpu.VMEM((16,), jnp.int32),
            pltpu.VMEM((128,), jnp.int32),
            pltpu.VMEM((NUM_REL * PADC,), jnp.float32),
            pltpu.SemaphoreType.DMA,
        ],
        compiler_params=_sc_params(),
    )
    return f(comb_part, rowmeta)


def _sc_aggregate(h, src_part, comb_part, rowmeta, zeros_rows, n):
    def body(h_hbm, srcp_hbm, combp_hbm, meta_hbm, z_hbm, agg_hbm,
             m16, zv, sidx0, cidx0, rows0, sidx1, cidx1, rows1,
             acc_sh, semg, sems):
        c = lax.axis_index("c")
        tid = lax.axis_index("s")
        pltpu.sync_copy(meta_hbm.at[0], m16)
        meta = m16[...]
        pltpu.sync_copy(z_hbm, zv)

        sidx = [sidx0, sidx1]
        cidx = [cidx0, cidx1]
        rows = [rows0, rows1]

        for cpass in range(SCP):
            b = c * SCP + cpass
            start = _extract(meta, b)
            end = _extract(meta, b + 1)

            z0 = tid * (NUM_REL * PADC // NTILE)
            for zi in range(9):
                pltpu.sync_copy(zv, acc_sh.at[pl.ds(z0 + zi * 128, 128), :])
            pltpu.sync_copy(zv.at[pl.ds(0, 48), :],
                            acc_sh.at[pl.ds(z0 + 9 * 128, 48), :])
            plsc.subcore_barrier()

            def cond2(k):
                return k + 16 < end

            def pair(k):
                pltpu.sync_copy(srcp_hbm.at[pl.ds(k * 128, 128)], sidx[0])
                pltpu.sync_copy(combp_hbm.at[pl.ds(k * 128, 128)], cidx[0])
                g0 = pltpu.async_copy(h_hbm.at[sidx[0]], rows[0], semg)
                k1 = k + 16
                pltpu.sync_copy(srcp_hbm.at[pl.ds(k1 * 128, 128)], sidx[1])
                pltpu.sync_copy(combp_hbm.at[pl.ds(k1 * 128, 128)], cidx[1])
                g1 = pltpu.async_copy(h_hbm.at[sidx[1]], rows[1], semg)
                g0.wait()
                s0 = pltpu.async_copy(rows[0], acc_sh.at[cidx[0]], sems,
                                      add=True)
                g1.wait()
                s1 = pltpu.async_copy(rows[1], acc_sh.at[cidx[1]], sems,
                                      add=True)
                s0.wait()
                s1.wait()
                return k + 32

            k = lax.while_loop(cond2, pair, start + tid)

            @pl.when(k < end)
            def _tail():
                pltpu.sync_copy(srcp_hbm.at[pl.ds(k * 128, 128)], sidx[0])
                pltpu.sync_copy(combp_hbm.at[pl.ds(k * 128, 128)], cidx[0])
                pltpu.async_copy(h_hbm.at[sidx[0]], rows[0], semg).wait()
                pltpu.async_copy(rows[0], acc_sh.at[cidx[0]], sems,
                                 add=True).wait()

            plsc.subcore_barrier()

            for r in range(NUM_REL):
                def dcond(m):
                    return m < NCK // 125

                def dbody(m):
                    pltpu.sync_copy(
                        acc_sh.at[pl.ds(r * PADC + m * 125, 125), :],
                        agg_hbm.at[r, pl.ds(b * NCK + m * 125, 125), :])
                    return m + 16

                lax.while_loop(dcond, dbody, tid)
            plsc.subcore_barrier()

    f = pl.kernel(
        body,
        out_type=jax.ShapeDtypeStruct((NUM_REL, n, 64), jnp.float32),
        mesh=_mesh(),
        scratch_types=[
            pltpu.VMEM((16,), jnp.int32),
            pltpu.VMEM((128, 64), jnp.float32),
            pltpu.VMEM((128,), jnp.int32),
            pltpu.VMEM((128,), jnp.int32),
            pltpu.VMEM((128, 64), jnp.float32),
            pltpu.VMEM((128,), jnp.int32),
            pltpu.VMEM((128,), jnp.int32),
            pltpu.VMEM((128, 64), jnp.float32),
            pltpu.VMEM_SHARED((NUM_REL * PADC, 64), jnp.float32),
            pltpu.SemaphoreType.DMA,
            pltpu.SemaphoreType.DMA,
        ],
        compiler_params=_sc_params(),
    )
    return f(h, src_part, comb_part, rowmeta, zeros_rows)


def _partition_glue(cnt_wb):
    cnt = cnt_wb[:, :NB]
    pc = (cnt + 7) & ~jnp.int32(7)
    u = jnp.sum(pc, axis=0)
    rows = (u + 255) // 128
    start_row = jnp.concatenate([jnp.zeros((1,), jnp.int32),
                                 jnp.cumsum(rows)]).astype(jnp.int32)
    start_ent = start_row[:NB] * 128
    slot_off = start_ent[None, :] + (jnp.cumsum(pc, axis=0) - pc)
    loc_off = jnp.cumsum(pc, axis=1) - pc
    dummy_a = start_ent + u
    dummy_b = start_row[1:] * 128 - 128

    def pad16(a):
        return jnp.concatenate(
            [a.astype(jnp.int32),
             jnp.zeros(a.shape[:-1] + (16 - a.shape[-1],), jnp.int32)], -1)

    rowmeta = pad16(start_row[None, :NB + 1])
    dummy_meta = jnp.stack([pad16(dummy_a[None, :])[0],
                            pad16(dummy_b[None, :])[0]])
    return pad16(slot_off), pad16(loc_off), rowmeta, dummy_meta


def kernel(x, edge_index, edge_type, batch, embed, W1, Wroot1, b1, W2, Wroot2, b2, linW, linb):
    n = x.shape[0]
    e = edge_index.shape[1]
    src, dst = edge_index[0], edge_index[1]

    npad = ((n + 128 * NW - 1) // (128 * NW)) * (128 * NW)
    x_pad = jnp.concatenate([x.astype(jnp.int32), jnp.zeros((npad - n,), jnp.int32)])
    h0 = _sc_embed_gather(x_pad, embed)

    epad = NW * ET
    src_p = jnp.concatenate([src.astype(jnp.int32), jnp.zeros((epad - e,), jnp.int32)])
    dst_p = jnp.concatenate([dst.astype(jnp.int32), jnp.full((epad - e,), n, jnp.int32)])
    typ_p = jnp.concatenate([edge_type.astype(jnp.int32), jnp.zeros((epad - e,), jnp.int32)])

    cnt_wb = _sc_count(dst_p)
    slot_off, loc_off, rowmeta, dummy_meta = _partition_glue(cnt_wb)
    src_part, comb_part = _sc_partition(src_p, dst_p, typ_p, slot_off, loc_off,
                                        cnt_wb, dummy_meta)
    hist = _sc_hist(comb_part, rowmeta)
    cnt = (hist.reshape(NB, 4, NUM_REL, PADC).sum(axis=1)[:, :, :NCK]
           .transpose(1, 0, 2).reshape(NUM_REL, n))
    icnt = (1.0 / jnp.maximum(cnt, 1.0)).T
    batch3d = batch.reshape(n // ROW_BLOCK, 1, ROW_BLOCK)
    zrows = jnp.zeros((128, 64), jnp.float32)

    agg1 = _sc_aggregate(h0, src_part, comb_part, rowmeta, zrows, n)
    h1 = _tc_dense(h0, agg1, icnt, Wroot1, b1, W1, n=n)
    agg2 = _sc_aggregate(h1, src_part, comb_part, rowmeta, zrows, n)
    h2 = _tc_dense(h1, agg2, icnt, Wroot2, b2, W2)
    return _tc_pool(h2, batch3d, linW, linb)


# speedup vs baseline: 12.9841x; 1.2455x over previous
import functools
import jax
import jax.numpy as jnp
from jax import lax
from jax.experimental import pallas as pl
from jax.experimental.pallas import tpu as pltpu
from jax.experimental.pallas import tpu_sc as plsc

NUM_REL = 3
NUM_GRAPHS = 64
ROW_BLOCK = 2000

NSC = 2
NTILE = 16
NW = NSC * NTILE


def _mesh():
    return plsc.VectorSubcoreMesh(core_axis_name="c", subcore_axis_name="s")


def _wid():
    return lax.axis_index("s") * NSC + lax.axis_index("c")


def _sc_embed_gather(x_pad, embed):
    npad, d = x_pad.shape[0], embed.shape[1]
    nch = npad // 128
    per_w = nch // NW

    def body(x_hbm, table_hbm, out_hbm, idx_v, rows_v, sem):
        w = _wid()

        def step(j, _):
            k = w * per_w + j
            pltpu.sync_copy(x_hbm.at[pl.ds(k * 128, 128)], idx_v)
            pltpu.async_copy(table_hbm.at[idx_v], rows_v, sem).wait()
            pltpu.sync_copy(rows_v, out_hbm.at[pl.ds(k * 128, 128), :])
            return ()

        lax.fori_loop(0, per_w, step, ())

    f = pl.kernel(
        body,
        out_type=jax.ShapeDtypeStruct((npad, d), jnp.float32),
        mesh=_mesh(),
        scratch_types=[
            pltpu.VMEM((128,), jnp.int32),
            pltpu.VMEM((128, d), jnp.float32),
            pltpu.SemaphoreType.DMA,
        ],
        compiler_params=pltpu.CompilerParams(use_tc_tiling_on_sc=False, needs_layout_passes=False),
    )
    return f(x_pad, embed)


def _dense_body(h_ref, agg_ref, icnt_ref, wroot_ref, b_ref, w_ref, out_ref):
    h = h_ref[...]
    acc = jnp.dot(h, wroot_ref[...], preferred_element_type=jnp.float32)
    acc = acc + b_ref[...][None, :]
    for r in range(NUM_REL):
        m = agg_ref[r] * icnt_ref[:, r][:, None]
        acc = acc + jnp.dot(m, w_ref[r], preferred_element_type=jnp.float32)
    out_ref[...] = jnp.maximum(acc, 0.0)


def _tc_dense(h, agg, icnt, wroot, b, w, n=None):
    d = h.shape[1]
    if n is None:
        n = h.shape[0]
    hh = w.shape[2]
    grid = n // ROW_BLOCK
    return pl.pallas_call(
        _dense_body,
        grid=(grid,),
        in_specs=[
            pl.BlockSpec((ROW_BLOCK, d), lambda i: (i, 0)),
            pl.BlockSpec((NUM_REL, ROW_BLOCK, d), lambda i: (0, i, 0)),
            pl.BlockSpec((ROW_BLOCK, NUM_REL), lambda i: (i, 0)),
            pl.BlockSpec((d, hh), lambda i: (0, 0)),
            pl.BlockSpec((hh,), lambda i: (0,)),
            pl.BlockSpec((NUM_REL, d, hh), lambda i: (0, 0, 0)),
        ],
        out_specs=pl.BlockSpec((ROW_BLOCK, hh), lambda i: (i, 0)),
        out_shape=jax.ShapeDtypeStruct((n, hh), jnp.float32),
    )(h, agg, icnt, wroot, b, w)


def _pool_body(h_ref, batch_ref, linw_ref, linb_ref, out_ref, acc_ref, cnt_ref):
    i = pl.program_id(0)

    @pl.when(i == 0)
    def _init():
        acc_ref[...] = jnp.zeros_like(acc_ref)
        cnt_ref[...] = jnp.zeros_like(cnt_ref)

    bvec = batch_ref[0, 0, :]
    iota = lax.broadcasted_iota(jnp.int32, (ROW_BLOCK, NUM_GRAPHS), 1)
    onehot = (bvec[:, None] == iota).astype(jnp.float32)
    acc_ref[...] += lax.dot_general(
        onehot, h_ref[...], (((0,), (0,)), ((), ())),
        preferred_element_type=jnp.float32)
    cnt_ref[...] += jnp.sum(onehot, axis=0, keepdims=True)

    @pl.when(i == pl.num_programs(0) - 1)
    def _fin():
        pooled = acc_ref[...] / jnp.maximum(cnt_ref[...], 1.0).T
        out_ref[...] = jnp.dot(pooled, linw_ref[...],
                               preferred_element_type=jnp.float32) + linb_ref[...][None, :]


def _tc_pool(h, batch3d, linw, linb):
    n, d = h.shape
    c = linw.shape[1]
    grid = n // ROW_BLOCK
    return pl.pallas_call(
        _pool_body,
        grid=(grid,),
        in_specs=[
            pl.BlockSpec((ROW_BLOCK, d), lambda i: (i, 0)),
            pl.BlockSpec((1, 1, ROW_BLOCK), lambda i: (i, 0, 0)),
            pl.BlockSpec((d, c), lambda i: (0, 0)),
            pl.BlockSpec((c,), lambda i: (0,)),
        ],
        out_specs=pl.BlockSpec((NUM_GRAPHS, c), lambda i: (0, 0)),
        out_shape=jax.ShapeDtypeStruct((NUM_GRAPHS, c), jnp.float32),
        scratch_shapes=[
            pltpu.VMEM((NUM_GRAPHS, d), jnp.float32),
            pltpu.VMEM((1, NUM_GRAPHS), jnp.float32),
        ],
    )(h, batch3d, linw, linb)


NB = 8
NCK = 6250
PADC = 6400
DUM = NCK
ET = 25600
CHK = 1600
CAPROWS = 6400
SCP = NB // NSC


def _iota16():
    return lax.iota(jnp.int32, 16)


def _extract(vec16, i):
    return jnp.sum(jnp.where(_iota16() == i, vec16, 0))


def _sc_params():
    return pltpu.CompilerParams(use_tc_tiling_on_sc=False, needs_layout_passes=False)


def _sc_count(dst_pad):
    def body(dst_hbm, out_hbm, chunk_v, row_v, sem):
        w = _wid()

        def chunk_body(ci, cnts):
            pltpu.sync_copy(dst_hbm.at[pl.ds(w * ET + ci * CHK, CHK)], chunk_v)

            def vec_body(vi, cnts):
                d = chunk_v[pl.ds(vi * 16, 16)]
                bkt = d // NCK
                return tuple(cnts[b] + (bkt == b).astype(jnp.int32)
                             for b in range(NB))

            return lax.fori_loop(0, CHK // 16, vec_body, cnts)

        cnts = lax.fori_loop(0, ET // CHK, chunk_body,
                             tuple(jnp.zeros((16,), jnp.int32) for _ in range(NB)))
        row = jnp.zeros((16,), jnp.int32)
        for b in range(NB):
            row = jnp.where(_iota16() == b, jnp.sum(cnts[b]), row)
        row_v[...] = row
        pltpu.sync_copy(row_v, out_hbm.at[w])

    f = pl.kernel(
        body,
        out_type=jax.ShapeDtypeStruct((NW, 16), jnp.int32),
        mesh=_mesh(),
        scratch_types=[
            pltpu.VMEM((CHK,), jnp.int32),
            pltpu.VMEM((16,), jnp.int32),
            pltpu.SemaphoreType.DMA,
        ],
        compiler_params=_sc_params(),
    )
    return f(dst_pad)


def _sc_partition(src_pad, dst_pad, typ_pad, slot_off, loc_off, cnt_wb, dummy_meta):
    STG = ET + NB * 16

    def body(src_hbm, dst_hbm, typ_hbm, slot_hbm, loc_hbm, cnt_hbm, dmy_hbm,
             srcp_hbm, combp_hbm,
             srcv, dstv, typv, sstage, cstage, m16, dzero, ddum, sem):
        w = _wid()

        pltpu.sync_copy(slot_hbm.at[w], m16)
        slot = m16[...]
        pltpu.sync_copy(loc_hbm.at[w], m16)
        loc = m16[...]
        pltpu.sync_copy(cnt_hbm.at[w], m16)
        cnt = m16[...]

        for i in range(8):
            dzero[pl.ds(i * 16, 16)] = jnp.zeros((16,), jnp.int32)
            ddum[pl.ds(i * 16, 16)] = jnp.full((16,), DUM, jnp.int32)

        @pl.when(w < NB)
        def _dummies():
            pltpu.sync_copy(dmy_hbm.at[0], m16)
            offa = pl.multiple_of(_extract(m16[...], w), 8)
            pltpu.sync_copy(dmy_hbm.at[1], m16)
            offb = pl.multiple_of(_extract(m16[...], w), 8)
            pltpu.sync_copy(dzero, srcp_hbm.at[pl.ds(offa, 128)])
            pltpu.sync_copy(dzero, srcp_hbm.at[pl.ds(offb, 128)])
            pltpu.sync_copy(ddum, combp_hbm.at[pl.ds(offa, 128)])
            pltpu.sync_copy(ddum, combp_hbm.at[pl.ds(offb, 128)])

        init = tuple(_extract(loc, b) for b in range(NB))

        def chunk_body(ci, cur):
            base = w * ET + ci * CHK
            pltpu.sync_copy(src_hbm.at[pl.ds(base, CHK)], srcv)
            pltpu.sync_copy(dst_hbm.at[pl.ds(base, CHK)], dstv)
            pltpu.sync_copy(typ_hbm.at[pl.ds(base, CHK)], typv)

            def vec_body(vi, cur):
                s = srcv[pl.ds(vi * 16, 16)]
                d = dstv[pl.ds(vi * 16, 16)]
                t = typv[pl.ds(vi * 16, 16)]
                bkt = d // NCK
                cb = t * PADC + (d - bkt * NCK)
                out = []
                for b in range(NB):
                    m = bkt == b
                    mi = m.astype(jnp.int32)
                    off = cur[b] + plsc.cumsum(mi) - 1
                    plsc.store_scatter(sstage, [off], s, mask=m)
                    plsc.store_scatter(cstage, [off], cb, mask=m)
                    out.append(cur[b] + jnp.sum(mi))
                return tuple(out)

            return lax.fori_loop(0, CHK // 16, vec_body, cur)

        cur = lax.fori_loop(0, ET // CHK, chunk_body, init)

        for b in range(NB):
            nb_cnt = _extract(cnt, b)
            pc = (nb_cnt + 7) & ~jnp.int32(7)
            toff = cur[b] + _iota16()
            tm = _iota16() < (pc - nb_cnt)
            plsc.store_scatter(sstage, [toff], jnp.zeros((16,), jnp.int32),
                               mask=tm)
            plsc.store_scatter(cstage, [toff], jnp.full((16,), DUM, jnp.int32),
                               mask=tm)
            lo = _extract(loc, b)
            go = _extract(slot, b)

            def drain(step, j0):
                def cond(j):
                    return j + step <= pc

                def dbody(j):
                    lj = pl.multiple_of(lo + j, 8)
                    gj = pl.multiple_of(go + j, 8)
                    pltpu.sync_copy(sstage.at[pl.ds(lj, step)],
                                    srcp_hbm.at[pl.ds(gj, step)])
                    pltpu.sync_copy(cstage.at[pl.ds(lj, step)],
                                    combp_hbm.at[pl.ds(gj, step)])
                    return j + step

                return lax.while_loop(cond, dbody, j0)

            j = drain(512, jnp.int32(0))
            j = drain(64, j)
            drain(8, j)

    f = pl.kernel(
        body,
        out_type=(jax.ShapeDtypeStruct((CAPROWS * 128,), jnp.int32),
                  jax.ShapeDtypeStruct((CAPROWS * 128,), jnp.int32)),
        mesh=_mesh(),
        scratch_types=[
            pltpu.VMEM((CHK,), jnp.int32),
            pltpu.VMEM((CHK,), jnp.int32),
            pltpu.VMEM((CHK,), jnp.int32),
            pltpu.VMEM((STG,), jnp.int32),
            pltpu.VMEM((STG,), jnp.int32),
            pltpu.VMEM((16,), jnp.int32),
            pltpu.VMEM((128,), jnp.int32),
            pltpu.VMEM((128,), jnp.int32),
            pltpu.SemaphoreType.DMA,
        ],
        compiler_params=_sc_params(),
    )
    return f(src_pad, dst_pad, typ_pad, slot_off, loc_off, cnt_wb, dummy_meta)


def _sc_hist(comb_part, rowmeta):
    def body(comb_hbm, meta_hbm, out_hbm, m16, cidx, hist, sem):
        w = _wid()
        b = w // 4
        pltpu.sync_copy(meta_hbm.at[0], m16)
        meta = m16[...]
        start = _extract(meta, b)
        end = _extract(meta, b + 1)

        def zbody(i, _):
            hist[pl.ds(i * 16, 16)] = jnp.zeros((16,), jnp.float32)
            return ()

        lax.fori_loop(0, NUM_REL * PADC // 16, zbody, ())

        def cond(k):
            return k < end

        def kbody(k):
            pltpu.sync_copy(comb_hbm.at[pl.ds(k * 128, 128)], cidx)

            def vbody(vi, _):
                c = cidx[pl.ds(vi * 16, 16)]
                plsc.addupdate_scatter(hist, [c], jnp.ones((16,), jnp.float32))
                return ()

            lax.fori_loop(0, 8, vbody, ())
            return k + 4

        lax.while_loop(cond, kbody, start + (w % 4))
        pltpu.sync_copy(hist, out_hbm.at[w])

    f = pl.kernel(
        body,
        out_type=jax.ShapeDtypeStruct((NW, NUM_REL * PADC), jnp.float32),
        mesh=_mesh(),
        scratch_types=[
            pltpu.VMEM((16,), jnp.int32),
            pltpu.VMEM((128,), jnp.int32),
            pltpu.VMEM((NUM_REL * PADC,), jnp.float32),
            pltpu.SemaphoreType.DMA,
        ],
        compiler_params=_sc_params(),
    )
    return f(comb_part, rowmeta)


def _sc_aggregate(h, src_part, comb_part, rowmeta, zeros_rows, n):
    DEPTH = 4

    def body(h_hbm, srcp_hbm, combp_hbm, meta_hbm, z_hbm, agg_hbm,
             m16, zv,
             sidx0, cidx0, rows0, sidx1, cidx1, rows1,
             sidx2, cidx2, rows2, sidx3, cidx3, rows3,
             acc_sh, semi, semg, sems):
        c = lax.axis_index("c")
        tid = lax.axis_index("s")
        pltpu.sync_copy(meta_hbm.at[0], m16)
        meta = m16[...]
        pltpu.sync_copy(z_hbm, zv)

        sidx = [sidx0, sidx1, sidx2, sidx3]
        cidx = [cidx0, cidx1, cidx2, cidx3]
        rows = [rows0, rows1, rows2, rows3]

        for cpass in range(SCP):
            b = c * SCP + cpass
            start = _extract(meta, b)
            end = _extract(meta, b + 1)

            z0 = tid * (NUM_REL * PADC // NTILE)
            for zi in range(9):
                pltpu.sync_copy(zv, acc_sh.at[pl.ds(z0 + zi * 128, 128), :])
            pltpu.sync_copy(zv.at[pl.ds(0, 48), :],
                            acc_sh.at[pl.ds(z0 + 9 * 128, 48), :])
            plsc.subcore_barrier()

            def condg(k):
                return k + 16 * (DEPTH - 1) < end

            def group(k):
                hi = []
                for s in range(DEPTH):
                    ks = k + 16 * s
                    hi.append(pltpu.async_copy(
                        srcp_hbm.at[pl.ds(ks * 128, 128)], sidx[s], semi))
                    hi.append(pltpu.async_copy(
                        combp_hbm.at[pl.ds(ks * 128, 128)], cidx[s], semi))
                for h in hi:
                    h.wait()
                hg = [pltpu.async_copy(h_hbm.at[sidx[s]], rows[s], semg)
                      for s in range(DEPTH)]
                ha = []
                for s in range(DEPTH):
                    hg[s].wait()
                    ha.append(pltpu.async_copy(rows[s], acc_sh.at[cidx[s]],
                                               sems, add=True))
                for h in ha:
                    h.wait()
                return k + 16 * DEPTH

            k = lax.while_loop(condg, group, start + tid)

            def condt(k):
                return k < end

            def tailc(k):
                pltpu.sync_copy(srcp_hbm.at[pl.ds(k * 128, 128)], sidx[0])
                pltpu.sync_copy(combp_hbm.at[pl.ds(k * 128, 128)], cidx[0])
                pltpu.async_copy(h_hbm.at[sidx[0]], rows[0], semg).wait()
                pltpu.async_copy(rows[0], acc_sh.at[cidx[0]], sems,
                                 add=True).wait()
                return k + 16

            lax.while_loop(condt, tailc, k)

            plsc.subcore_barrier()

            for r in range(NUM_REL):
                def dcond(m):
                    return m < NCK // 125

                def dbody(m):
                    pltpu.sync_copy(
                        acc_sh.at[pl.ds(r * PADC + m * 125, 125), :],
                        agg_hbm.at[r, pl.ds(b * NCK + m * 125, 125), :])
                    return m + 16

                lax.while_loop(dcond, dbody, tid)
            plsc.subcore_barrier()

    f = pl.kernel(
        body,
        out_type=jax.ShapeDtypeStruct((NUM_REL, n, 64), jnp.float32),
        mesh=_mesh(),
        scratch_types=[
            pltpu.VMEM((16,), jnp.int32),
            pltpu.VMEM((128, 64), jnp.float32),
        ] + [
            t
            for _ in range(DEPTH)
            for t in (pltpu.VMEM((128,), jnp.int32),
                      pltpu.VMEM((128,), jnp.int32),
                      pltpu.VMEM((128, 64), jnp.float32))
        ] + [
            pltpu.VMEM_SHARED((NUM_REL * PADC, 64), jnp.float32),
            pltpu.SemaphoreType.DMA,
            pltpu.SemaphoreType.DMA,
            pltpu.SemaphoreType.DMA,
        ],
        compiler_params=_sc_params(),
    )
    return f(h, src_part, comb_part, rowmeta, zeros_rows)


def _partition_glue(cnt_wb):
    cnt = cnt_wb[:, :NB]
    pc = (cnt + 7) & ~jnp.int32(7)
    u = jnp.sum(pc, axis=0)
    rows = (u + 255) // 128
    start_row = jnp.concatenate([jnp.zeros((1,), jnp.int32),
                                 jnp.cumsum(rows)]).astype(jnp.int32)
    start_ent = start_row[:NB] * 128
    slot_off = start_ent[None, :] + (jnp.cumsum(pc, axis=0) - pc)
    loc_off = jnp.cumsum(pc, axis=1) - pc
    dummy_a = start_ent + u
    dummy_b = start_row[1:] * 128 - 128

    def pad16(a):
        return jnp.concatenate(
            [a.astype(jnp.int32),
             jnp.zeros(a.shape[:-1] + (16 - a.shape[-1],), jnp.int32)], -1)

    rowmeta = pad16(start_row[None, :NB + 1])
    dummy_meta = jnp.stack([pad16(dummy_a[None, :])[0],
                            pad16(dummy_b[None, :])[0]])
    return pad16(slot_off), pad16(loc_off), rowmeta, dummy_meta


def kernel(x, edge_index, edge_type, batch, embed, W1, Wroot1, b1, W2, Wroot2, b2, linW, linb):
    n = x.shape[0]
    e = edge_index.shape[1]
    src, dst = edge_index[0], edge_index[1]

    npad = ((n + 128 * NW - 1) // (128 * NW)) * (128 * NW)
    x_pad = jnp.concatenate([x.astype(jnp.int32), jnp.zeros((npad - n,), jnp.int32)])
    h0 = _sc_embed_gather(x_pad, embed)

    epad = NW * ET
    src_p = jnp.concatenate([src.astype(jnp.int32), jnp.zeros((epad - e,), jnp.int32)])
    dst_p = jnp.concatenate([dst.astype(jnp.int32), jnp.full((epad - e,), n, jnp.int32)])
    typ_p = jnp.concatenate([edge_type.astype(jnp.int32), jnp.zeros((epad - e,), jnp.int32)])

    cnt_wb = _sc_count(dst_p)
    slot_off, loc_off, rowmeta, dummy_meta = _partition_glue(cnt_wb)
    src_part, comb_part = _sc_partition(src_p, dst_p, typ_p, slot_off, loc_off,
                                        cnt_wb, dummy_meta)
    hist = _sc_hist(comb_part, rowmeta)
    cnt = (hist.reshape(NB, 4, NUM_REL, PADC).sum(axis=1)[:, :, :NCK]
           .transpose(1, 0, 2).reshape(NUM_REL, n))
    icnt = (1.0 / jnp.maximum(cnt, 1.0)).T
    batch3d = batch.reshape(n // ROW_BLOCK, 1, ROW_BLOCK)
    zrows = jnp.zeros((128, 64), jnp.float32)

    agg1 = _sc_aggregate(h0, src_part, comb_part, rowmeta, zrows, n)
    h1 = _tc_dense(h0, agg1, icnt, Wroot1, b1, W1, n=n)
    agg2 = _sc_aggregate(h1, src_part, comb_part, rowmeta, zrows, n)
    h2 = _tc_dense(h1, agg2, icnt, Wroot2, b2, W2)
    return _tc_pool(h2, batch3d, linW, linb)


# speedup vs baseline: 14.2379x; 1.0966x over previous
import functools
import jax
import jax.numpy as jnp
from jax import lax
from jax.experimental import pallas as pl
from jax.experimental.pallas import tpu as pltpu
from jax.experimental.pallas import tpu_sc as plsc

NUM_REL = 3
NUM_GRAPHS = 64
ROW_BLOCK = 2000

NSC = 2
NTILE = 16
NW = NSC * NTILE


def _mesh():
    return plsc.VectorSubcoreMesh(core_axis_name="c", subcore_axis_name="s")


def _wid():
    return lax.axis_index("s") * NSC + lax.axis_index("c")


def _sc_embed_gather(x_pad, embed):
    npad, d = x_pad.shape[0], embed.shape[1]
    nch = npad // 128
    per_w = nch // NW

    def body(x_hbm, table_hbm, out_hbm, idx0, idx1, rows0, rows1,
             semi, semg, semw):
        w = _wid()
        idx = [idx0, idx1]
        rows = [rows0, rows1]
        pend_w = [None, None]
        pend_i = pltpu.async_copy(x_hbm.at[pl.ds(w * per_w * 128, 128)],
                                  idx0, semi)
        for j in range(per_w):
            k = w * per_w + j
            pend_i.wait()
            if j + 1 < per_w:
                pend_i = pltpu.async_copy(
                    x_hbm.at[pl.ds((k + 1) * 128, 128)], idx[(j + 1) % 2],
                    semi)
            if pend_w[j % 2] is not None:
                pend_w[j % 2].wait()
            pltpu.async_copy(table_hbm.at[idx[j % 2]], rows[j % 2],
                             semg).wait()
            pend_w[j % 2] = pltpu.async_copy(
                rows[j % 2], out_hbm.at[pl.ds(k * 128, 128), :], semw)
        for p in pend_w:
            if p is not None:
                p.wait()

    f = pl.kernel(
        body,
        out_type=jax.ShapeDtypeStruct((npad, d), jnp.float32),
        mesh=_mesh(),
        scratch_types=[
            pltpu.VMEM((128,), jnp.int32),
            pltpu.VMEM((128,), jnp.int32),
            pltpu.VMEM((128, d), jnp.float32),
            pltpu.VMEM((128, d), jnp.float32),
            pltpu.SemaphoreType.DMA,
            pltpu.SemaphoreType.DMA,
            pltpu.SemaphoreType.DMA,
        ],
        compiler_params=pltpu.CompilerParams(use_tc_tiling_on_sc=False, needs_layout_passes=False),
    )
    return f(x_pad, embed)


def _dense_body(h_ref, agg_ref, icnt_ref, wroot_ref, b_ref, w_ref, out_ref):
    h = h_ref[...]
    acc = jnp.dot(h, wroot_ref[...], preferred_element_type=jnp.float32)
    acc = acc + b_ref[...][None, :]
    for r in range(NUM_REL):
        m = agg_ref[r] * icnt_ref[:, r][:, None]
        acc = acc + jnp.dot(m, w_ref[r], preferred_element_type=jnp.float32)
    out_ref[...] = jnp.maximum(acc, 0.0)


def _tc_dense(h, agg, icnt, wroot, b, w, n=None):
    d = h.shape[1]
    if n is None:
        n = h.shape[0]
    hh = w.shape[2]
    grid = n // ROW_BLOCK
    return pl.pallas_call(
        _dense_body,
        grid=(grid,),
        in_specs=[
            pl.BlockSpec((ROW_BLOCK, d), lambda i: (i, 0)),
            pl.BlockSpec((NUM_REL, ROW_BLOCK, d), lambda i: (0, i, 0)),
            pl.BlockSpec((ROW_BLOCK, NUM_REL), lambda i: (i, 0)),
            pl.BlockSpec((d, hh), lambda i: (0, 0)),
            pl.BlockSpec((hh,), lambda i: (0,)),
            pl.BlockSpec((NUM_REL, d, hh), lambda i: (0, 0, 0)),
        ],
        out_specs=pl.BlockSpec((ROW_BLOCK, hh), lambda i: (i, 0)),
        out_shape=jax.ShapeDtypeStruct((n, hh), jnp.float32),
    )(h, agg, icnt, wroot, b, w)


def _pool_body(h_ref, batch_ref, linw_ref, linb_ref, out_ref, acc_ref, cnt_ref):
    i = pl.program_id(0)

    @pl.when(i == 0)
    def _init():
        acc_ref[...] = jnp.zeros_like(acc_ref)
        cnt_ref[...] = jnp.zeros_like(cnt_ref)

    bvec = batch_ref[0, 0, :]
    iota = lax.broadcasted_iota(jnp.int32, (ROW_BLOCK, NUM_GRAPHS), 1)
    onehot = (bvec[:, None] == iota).astype(jnp.float32)
    acc_ref[...] += lax.dot_general(
        onehot, h_ref[...], (((0,), (0,)), ((), ())),
        preferred_element_type=jnp.float32)
    cnt_ref[...] += jnp.sum(onehot, axis=0, keepdims=True)

    @pl.when(i == pl.num_programs(0) - 1)
    def _fin():
        pooled = acc_ref[...] / jnp.maximum(cnt_ref[...], 1.0).T
        out_ref[...] = jnp.dot(pooled, linw_ref[...],
                               preferred_element_type=jnp.float32) + linb_ref[...][None, :]


def _tc_pool(h, batch3d, linw, linb):
    n, d = h.shape
    c = linw.shape[1]
    grid = n // ROW_BLOCK
    return pl.pallas_call(
        _pool_body,
        grid=(grid,),
        in_specs=[
            pl.BlockSpec((ROW_BLOCK, d), lambda i: (i, 0)),
            pl.BlockSpec((1, 1, ROW_BLOCK), lambda i: (i, 0, 0)),
            pl.BlockSpec((d, c), lambda i: (0, 0)),
            pl.BlockSpec((c,), lambda i: (0,)),
        ],
        out_specs=pl.BlockSpec((NUM_GRAPHS, c), lambda i: (0, 0)),
        out_shape=jax.ShapeDtypeStruct((NUM_GRAPHS, c), jnp.float32),
        scratch_shapes=[
            pltpu.VMEM((NUM_GRAPHS, d), jnp.float32),
            pltpu.VMEM((1, NUM_GRAPHS), jnp.float32),
        ],
    )(h, batch3d, linw, linb)


NB = 8
NCK = 6250
PADC = 6400
DUM = NCK
ET = 25600
CHK = 1600
CAPROWS = 6400
SCP = NB // NSC


def _iota16():
    return lax.iota(jnp.int32, 16)


def _extract(vec16, i):
    return jnp.sum(jnp.where(_iota16() == i, vec16, 0))


def _sc_params():
    return pltpu.CompilerParams(use_tc_tiling_on_sc=False, needs_layout_passes=False)


def _sc_count(dst_pad):
    NCH = ET // CHK

    def body(dst_hbm, out_hbm, chunk0, chunk1, row_v, sem):
        w = _wid()
        bufs = [chunk0, chunk1]
        pend = pltpu.async_copy(dst_hbm.at[pl.ds(w * ET, CHK)], chunk0, sem)
        cnts = tuple(jnp.zeros((16,), jnp.int32) for _ in range(NB))
        for ci in range(NCH):
            pend.wait()
            if ci + 1 < NCH:
                pend = pltpu.async_copy(
                    dst_hbm.at[pl.ds(w * ET + (ci + 1) * CHK, CHK)],
                    bufs[(ci + 1) % 2], sem)
            buf = bufs[ci % 2]

            def vec_body(vi, cnts):
                d = buf[pl.ds(vi * 16, 16)]
                bkt = d // NCK
                return tuple(cnts[b] + (bkt == b).astype(jnp.int32)
                             for b in range(NB))

            cnts = lax.fori_loop(0, CHK // 16, vec_body, cnts)
        row = jnp.zeros((16,), jnp.int32)
        for b in range(NB):
            row = jnp.where(_iota16() == b, jnp.sum(cnts[b]), row)
        row_v[...] = row
        pltpu.sync_copy(row_v, out_hbm.at[w])

    f = pl.kernel(
        body,
        out_type=jax.ShapeDtypeStruct((NW, 16), jnp.int32),
        mesh=_mesh(),
        scratch_types=[
            pltpu.VMEM((CHK,), jnp.int32),
            pltpu.VMEM((CHK,), jnp.int32),
            pltpu.VMEM((16,), jnp.int32),
            pltpu.SemaphoreType.DMA,
        ],
        compiler_params=_sc_params(),
    )
    return f(dst_pad)


def _sc_partition(src_pad, dst_pad, typ_pad, slot_off, loc_off, cnt_wb, dummy_meta):
    STG = ET + NB * 16

    def body(src_hbm, dst_hbm, typ_hbm, slot_hbm, loc_hbm, cnt_hbm, dmy_hbm,
             srcp_hbm, combp_hbm,
             srcv0, dstv0, typv0, srcv1, dstv1, typv1,
             sstage, cstage, m16, dzero, ddum, semc, sem):
        w = _wid()
        srcb = [srcv0, srcv1]
        dstb = [dstv0, dstv1]
        typb = [typv0, typv1]

        pltpu.sync_copy(slot_hbm.at[w], m16)
        slot = m16[...]
        pltpu.sync_copy(loc_hbm.at[w], m16)
        loc = m16[...]
        pltpu.sync_copy(cnt_hbm.at[w], m16)
        cnt = m16[...]

        for i in range(8):
            dzero[pl.ds(i * 16, 16)] = jnp.zeros((16,), jnp.int32)
            ddum[pl.ds(i * 16, 16)] = jnp.full((16,), DUM, jnp.int32)

        @pl.when(w < NB)
        def _dummies():
            pltpu.sync_copy(dmy_hbm.at[0], m16)
            offa = pl.multiple_of(_extract(m16[...], w), 8)
            pltpu.sync_copy(dmy_hbm.at[1], m16)
            offb = pl.multiple_of(_extract(m16[...], w), 8)
            pltpu.sync_copy(dzero, srcp_hbm.at[pl.ds(offa, 128)])
            pltpu.sync_copy(dzero, srcp_hbm.at[pl.ds(offb, 128)])
            pltpu.sync_copy(ddum, combp_hbm.at[pl.ds(offa, 128)])
            pltpu.sync_copy(ddum, combp_hbm.at[pl.ds(offb, 128)])

        cur = tuple(_extract(loc, b) for b in range(NB))
        NCH = ET // CHK

        def load(ci, which):
            base = w * ET + ci * CHK
            return [pltpu.async_copy(src_hbm.at[pl.ds(base, CHK)],
                                     srcb[which], semc),
                    pltpu.async_copy(dst_hbm.at[pl.ds(base, CHK)],
                                     dstb[which], semc),
                    pltpu.async_copy(typ_hbm.at[pl.ds(base, CHK)],
                                     typb[which], semc)]

        pend = load(0, 0)
        for ci in range(NCH):
            for h in pend:
                h.wait()
            if ci + 1 < NCH:
                pend = load(ci + 1, (ci + 1) % 2)
            srcv, dstv, typv = srcb[ci % 2], dstb[ci % 2], typb[ci % 2]

            def vec_body(vi, cur, srcv=srcv, dstv=dstv, typv=typv):
                s = srcv[pl.ds(vi * 16, 16)]
                d = dstv[pl.ds(vi * 16, 16)]
                t = typv[pl.ds(vi * 16, 16)]
                bkt = d // NCK
                cb = t * PADC + (d - bkt * NCK)
                out = []
                for b in range(NB):
                    m = bkt == b
                    mi = m.astype(jnp.int32)
                    off = cur[b] + plsc.cumsum(mi) - 1
                    plsc.store_scatter(sstage, [off], s, mask=m)
                    plsc.store_scatter(cstage, [off], cb, mask=m)
                    out.append(cur[b] + jnp.sum(mi))
                return tuple(out)

            cur = lax.fori_loop(0, CHK // 16, vec_body, cur)

        for b in range(NB):
            nb_cnt = _extract(cnt, b)
            pc = (nb_cnt + 7) & ~jnp.int32(7)
            toff = cur[b] + _iota16()
            tm = _iota16() < (pc - nb_cnt)
            plsc.store_scatter(sstage, [toff], jnp.zeros((16,), jnp.int32),
                               mask=tm)
            plsc.store_scatter(cstage, [toff], jnp.full((16,), DUM, jnp.int32),
                               mask=tm)
            lo = _extract(loc, b)
            go = _extract(slot, b)

            def drain(step, j0):
                def cond(j):
                    return j + step <= pc

                def dbody(j):
                    lj = pl.multiple_of(lo + j, 8)
                    gj = pl.multiple_of(go + j, 8)
                    pltpu.sync_copy(sstage.at[pl.ds(lj, step)],
                                    srcp_hbm.at[pl.ds(gj, step)])
                    pltpu.sync_copy(cstage.at[pl.ds(lj, step)],
                                    combp_hbm.at[pl.ds(gj, step)])
                    return j + step

                return lax.while_loop(cond, dbody, j0)

            j = drain(512, jnp.int32(0))
            j = drain(64, j)
            drain(8, j)

    f = pl.kernel(
        body,
        out_type=(jax.ShapeDtypeStruct((CAPROWS * 128,), jnp.int32),
                  jax.ShapeDtypeStruct((CAPROWS * 128,), jnp.int32)),
        mesh=_mesh(),
        scratch_types=[
            pltpu.VMEM((CHK,), jnp.int32),
            pltpu.VMEM((CHK,), jnp.int32),
            pltpu.VMEM((CHK,), jnp.int32),
            pltpu.VMEM((CHK,), jnp.int32),
            pltpu.VMEM((CHK,), jnp.int32),
            pltpu.VMEM((CHK,), jnp.int32),
            pltpu.VMEM((STG,), jnp.int32),
            pltpu.VMEM((STG,), jnp.int32),
            pltpu.VMEM((16,), jnp.int32),
            pltpu.VMEM((128,), jnp.int32),
            pltpu.VMEM((128,), jnp.int32),
            pltpu.SemaphoreType.DMA,
            pltpu.SemaphoreType.DMA,
        ],
        compiler_params=_sc_params(),
    )
    return f(src_pad, dst_pad, typ_pad, slot_off, loc_off, cnt_wb, dummy_meta)


def _sc_hist(comb_part, rowmeta):
    GD = 4

    def body(comb_hbm, meta_hbm, out_hbm, m16, c0, c1, c2, c3, hist, sem):
        w = _wid()
        b = w // 4
        cb = [c0, c1, c2, c3]
        pltpu.sync_copy(meta_hbm.at[0], m16)
        meta = m16[...]
        start = _extract(meta, b)
        end = _extract(meta, b + 1)

        def zbody(i, _):
            hist[pl.ds(i * 16, 16)] = jnp.zeros((16,), jnp.float32)
            return ()

        lax.fori_loop(0, NUM_REL * PADC // 16, zbody, ())

        def scat(buf):
            def vbody(vi, _):
                c = buf[pl.ds(vi * 16, 16)]
                plsc.addupdate_scatter(hist, [c], jnp.ones((16,), jnp.float32))
                return ()

            lax.fori_loop(0, 8, vbody, ())

        def gcond(k):
            return k + 4 * (GD - 1) < end

        def gbody(k):
            hs = [pltpu.async_copy(
                comb_hbm.at[pl.ds((k + 4 * s) * 128, 128)], cb[s], sem)
                for s in range(GD)]
            for s in range(GD):
                hs[s].wait()
                scat(cb[s])
            return k + 4 * GD

        k = lax.while_loop(gcond, gbody, start + (w % 4))

        def cond(k):
            return k < end

        def kbody(k):
            pltpu.sync_copy(comb_hbm.at[pl.ds(k * 128, 128)], cb[0])
            scat(cb[0])
            return k + 4

        lax.while_loop(cond, kbody, k)
        pltpu.sync_copy(hist, out_hbm.at[w])

    f = pl.kernel(
        body,
        out_type=jax.ShapeDtypeStruct((NW, NUM_REL * PADC), jnp.float32),
        mesh=_mesh(),
        scratch_types=[
            pltpu.VMEM((16,), jnp.int32),
            pltpu.VMEM((128,), jnp.int32),
            pltpu.VMEM((128,), jnp.int32),
            pltpu.VMEM((128,), jnp.int32),
            pltpu.VMEM((128,), jnp.int32),
            pltpu.VMEM((NUM_REL * PADC,), jnp.float32),
            pltpu.SemaphoreType.DMA,
        ],
        compiler_params=_sc_params(),
    )
    return f(comb_part, rowmeta)


def _sc_aggregate(h, src_part, comb_part, rowmeta, zeros_rows, n):
    DEPTH = 5

    def body(h_hbm, srcp_hbm, combp_hbm, meta_hbm, z_hbm, agg_hbm,
             m16, zv, *rest):
        acc_sh, semi, semg, sems = rest[3 * DEPTH:]
        sidx = [rest[3 * s] for s in range(DEPTH)]
        cidx = [rest[3 * s + 1] for s in range(DEPTH)]
        rows = [rest[3 * s + 2] for s in range(DEPTH)]
        c = lax.axis_index("c")
        tid = lax.axis_index("s")
        pltpu.sync_copy(meta_hbm.at[0], m16)
        meta = m16[...]
        pltpu.sync_copy(z_hbm, zv)

        for cpass in range(SCP):
            b = c * SCP + cpass
            start = _extract(meta, b)
            end = _extract(meta, b + 1)

            z0 = tid * (NUM_REL * PADC // NTILE)
            zh = [pltpu.async_copy(
                zv, acc_sh.at[pl.ds(z0 + zi * 128, 128), :], semi)
                for zi in range(9)]
            zh.append(pltpu.async_copy(
                zv.at[pl.ds(0, 48), :],
                acc_sh.at[pl.ds(z0 + 9 * 128, 48), :], semi))
            for h in zh:
                h.wait()
            plsc.subcore_barrier()

            def condg(k):
                return k + 16 * (DEPTH - 1) < end

            def group(k):
                hi = []
                for s in range(DEPTH):
                    ks = k + 16 * s
                    hi.append(pltpu.async_copy(
                        srcp_hbm.at[pl.ds(ks * 128, 128)], sidx[s], semi))
                    hi.append(pltpu.async_copy(
                        combp_hbm.at[pl.ds(ks * 128, 128)], cidx[s], semi))
                for h in hi:
                    h.wait()
                hg = [pltpu.async_copy(h_hbm.at[sidx[s]], rows[s], semg)
                      for s in range(DEPTH)]
                ha = []
                for s in range(DEPTH):
                    hg[s].wait()
                    ha.append(pltpu.async_copy(rows[s], acc_sh.at[cidx[s]],
                                               sems, add=True))
                for h in ha:
                    h.wait()
                return k + 16 * DEPTH

            k = lax.while_loop(condg, group, start + tid)

            def condt(k):
                return k < end

            def tailc(k):
                pltpu.sync_copy(srcp_hbm.at[pl.ds(k * 128, 128)], sidx[0])
                pltpu.sync_copy(combp_hbm.at[pl.ds(k * 128, 128)], cidx[0])
                pltpu.async_copy(h_hbm.at[sidx[0]], rows[0], semg).wait()
                pltpu.async_copy(rows[0], acc_sh.at[cidx[0]], sems,
                                 add=True).wait()
                return k + 16

            lax.while_loop(condt, tailc, k)

            plsc.subcore_barrier()

            for r in range(NUM_REL):
                def dcond(m):
                    return m < NCK // 125

                def dbody(m):
                    pltpu.sync_copy(
                        acc_sh.at[pl.ds(r * PADC + m * 125, 125), :],
                        agg_hbm.at[r, pl.ds(b * NCK + m * 125, 125), :])
                    return m + 16

                lax.while_loop(dcond, dbody, tid)
            plsc.subcore_barrier()

    f = pl.kernel(
        body,
        out_type=jax.ShapeDtypeStruct((NUM_REL, n, 64), jnp.float32),
        mesh=_mesh(),
        scratch_types=[
            pltpu.VMEM((16,), jnp.int32),
            pltpu.VMEM((128, 64), jnp.float32),
        ] + [
            t
            for _ in range(DEPTH)
            for t in (pltpu.VMEM((128,), jnp.int32),
                      pltpu.VMEM((128,), jnp.int32),
                      pltpu.VMEM((128, 64), jnp.float32))
        ] + [
            pltpu.VMEM_SHARED((NUM_REL * PADC, 64), jnp.float32),
            pltpu.SemaphoreType.DMA,
            pltpu.SemaphoreType.DMA,
            pltpu.SemaphoreType.DMA,
        ],
        compiler_params=_sc_params(),
    )
    return f(h, src_part, comb_part, rowmeta, zeros_rows)


def _partition_glue(cnt_wb):
    cnt = cnt_wb[:, :NB]
    pc = (cnt + 7) & ~jnp.int32(7)
    u = jnp.sum(pc, axis=0)
    rows = (u + 255) // 128
    start_row = jnp.concatenate([jnp.zeros((1,), jnp.int32),
                                 jnp.cumsum(rows)]).astype(jnp.int32)
    start_ent = start_row[:NB] * 128
    slot_off = start_ent[None, :] + (jnp.cumsum(pc, axis=0) - pc)
    loc_off = jnp.cumsum(pc, axis=1) - pc
    dummy_a = start_ent + u
    dummy_b = start_row[1:] * 128 - 128

    def pad16(a):
        return jnp.concatenate(
            [a.astype(jnp.int32),
             jnp.zeros(a.shape[:-1] + (16 - a.shape[-1],), jnp.int32)], -1)

    rowmeta = pad16(start_row[None, :NB + 1])
    dummy_meta = jnp.stack([pad16(dummy_a[None, :])[0],
                            pad16(dummy_b[None, :])[0]])
    return pad16(slot_off), pad16(loc_off), rowmeta, dummy_meta


def kernel(x, edge_index, edge_type, batch, embed, W1, Wroot1, b1, W2, Wroot2, b2, linW, linb):
    n = x.shape[0]
    e = edge_index.shape[1]
    src, dst = edge_index[0], edge_index[1]

    npad = ((n + 128 * NW - 1) // (128 * NW)) * (128 * NW)
    x_pad = jnp.concatenate([x.astype(jnp.int32), jnp.zeros((npad - n,), jnp.int32)])
    h0 = _sc_embed_gather(x_pad, embed)

    epad = NW * ET
    src_p = jnp.concatenate([src.astype(jnp.int32), jnp.zeros((epad - e,), jnp.int32)])
    dst_p = jnp.concatenate([dst.astype(jnp.int32), jnp.full((epad - e,), n, jnp.int32)])
    typ_p = jnp.concatenate([edge_type.astype(jnp.int32), jnp.zeros((epad - e,), jnp.int32)])

    cnt_wb = _sc_count(dst_p)
    slot_off, loc_off, rowmeta, dummy_meta = _partition_glue(cnt_wb)
    src_part, comb_part = _sc_partition(src_p, dst_p, typ_p, slot_off, loc_off,
                                        cnt_wb, dummy_meta)
    hist = _sc_hist(comb_part, rowmeta)
    cnt = (hist.reshape(NB, 4, NUM_REL, PADC).sum(axis=1)[:, :, :NCK]
           .transpose(1, 0, 2).reshape(NUM_REL, n))
    icnt = (1.0 / jnp.maximum(cnt, 1.0)).T
    batch3d = batch.reshape(n // ROW_BLOCK, 1, ROW_BLOCK)
    zrows = jnp.zeros((128, 64), jnp.float32)

    agg1 = _sc_aggregate(h0, src_part, comb_part, rowmeta, zrows, n)
    h1 = _tc_dense(h0, agg1, icnt, Wroot1, b1, W1, n=n)
    agg2 = _sc_aggregate(h1, src_part, comb_part, rowmeta, zrows, n)
    h2 = _tc_dense(h1, agg2, icnt, Wroot2, b2, W2)
    return _tc_pool(h2, batch3d, linW, linb)


# speedup vs baseline: 14.5777x; 1.0239x over previous
import functools
import jax
import jax.numpy as jnp
from jax import lax
from jax.experimental import pallas as pl
from jax.experimental.pallas import tpu as pltpu
from jax.experimental.pallas import tpu_sc as plsc

NUM_REL = 3
NUM_GRAPHS = 64
ROW_BLOCK = 2000

NSC = 2
NTILE = 16
NW = NSC * NTILE


def _mesh():
    return plsc.VectorSubcoreMesh(core_axis_name="c", subcore_axis_name="s")


def _wid():
    return lax.axis_index("s") * NSC + lax.axis_index("c")


def _sc_embed_gather(x_pad, embed):
    npad, d = x_pad.shape[0], embed.shape[1]
    nch = npad // 128
    per_w = nch // NW

    def body(x_hbm, table_hbm, out_hbm, idx0, idx1, rows0, rows1,
             semi, semg, semw):
        w = _wid()
        idx = [idx0, idx1]
        rows = [rows0, rows1]
        pend_w = [None, None]
        pend_i = pltpu.async_copy(x_hbm.at[pl.ds(w * per_w * 128, 128)],
                                  idx0, semi)
        for j in range(per_w):
            k = w * per_w + j
            pend_i.wait()
            if j + 1 < per_w:
                pend_i = pltpu.async_copy(
                    x_hbm.at[pl.ds((k + 1) * 128, 128)], idx[(j + 1) % 2],
                    semi)
            if pend_w[j % 2] is not None:
                pend_w[j % 2].wait()
            pltpu.async_copy(table_hbm.at[idx[j % 2]], rows[j % 2],
                             semg).wait()
            pend_w[j % 2] = pltpu.async_copy(
                rows[j % 2], out_hbm.at[pl.ds(k * 128, 128), :], semw)
        for p in pend_w:
            if p is not None:
                p.wait()

    f = pl.kernel(
        body,
        out_type=jax.ShapeDtypeStruct((npad, d), jnp.float32),
        mesh=_mesh(),
        scratch_types=[
            pltpu.VMEM((128,), jnp.int32),
            pltpu.VMEM((128,), jnp.int32),
            pltpu.VMEM((128, d), jnp.float32),
            pltpu.VMEM((128, d), jnp.float32),
            pltpu.SemaphoreType.DMA,
            pltpu.SemaphoreType.DMA,
            pltpu.SemaphoreType.DMA,
        ],
        compiler_params=pltpu.CompilerParams(use_tc_tiling_on_sc=False, needs_layout_passes=False),
    )
    return f(x_pad, embed)


def _dense_body(h_ref, agg_ref, icnt_ref, wroot_ref, b_ref, w_ref, out_ref):
    h = h_ref[...]
    acc = jnp.dot(h, wroot_ref[...], preferred_element_type=jnp.float32)
    acc = acc + b_ref[...][None, :]
    for r in range(NUM_REL):
        m = agg_ref[r] * icnt_ref[:, r][:, None]
        acc = acc + jnp.dot(m, w_ref[r], preferred_element_type=jnp.float32)
    out_ref[...] = jnp.maximum(acc, 0.0)


def _tc_dense(h, agg, icnt, wroot, b, w, n=None):
    d = h.shape[1]
    if n is None:
        n = h.shape[0]
    hh = w.shape[2]
    grid = n // ROW_BLOCK
    return pl.pallas_call(
        _dense_body,
        grid=(grid,),
        in_specs=[
            pl.BlockSpec((ROW_BLOCK, d), lambda i: (i, 0)),
            pl.BlockSpec((NUM_REL, ROW_BLOCK, d), lambda i: (0, i, 0)),
            pl.BlockSpec((ROW_BLOCK, NUM_REL), lambda i: (i, 0)),
            pl.BlockSpec((d, hh), lambda i: (0, 0)),
            pl.BlockSpec((hh,), lambda i: (0,)),
            pl.BlockSpec((NUM_REL, d, hh), lambda i: (0, 0, 0)),
        ],
        out_specs=pl.BlockSpec((ROW_BLOCK, hh), lambda i: (i, 0)),
        out_shape=jax.ShapeDtypeStruct((n, hh), jnp.float32),
    )(h, agg, icnt, wroot, b, w)


def _pool_body(h_ref, batch_ref, linw_ref, linb_ref, out_ref, acc_ref, cnt_ref):
    i = pl.program_id(0)

    @pl.when(i == 0)
    def _init():
        acc_ref[...] = jnp.zeros_like(acc_ref)
        cnt_ref[...] = jnp.zeros_like(cnt_ref)

    bvec = batch_ref[0, 0, :]
    iota = lax.broadcasted_iota(jnp.int32, (ROW_BLOCK, NUM_GRAPHS), 1)
    onehot = (bvec[:, None] == iota).astype(jnp.float32)
    acc_ref[...] += lax.dot_general(
        onehot, h_ref[...], (((0,), (0,)), ((), ())),
        preferred_element_type=jnp.float32)
    cnt_ref[...] += jnp.sum(onehot, axis=0, keepdims=True)

    @pl.when(i == pl.num_programs(0) - 1)
    def _fin():
        pooled = acc_ref[...] / jnp.maximum(cnt_ref[...], 1.0).T
        out_ref[...] = jnp.dot(pooled, linw_ref[...],
                               preferred_element_type=jnp.float32) + linb_ref[...][None, :]


def _tc_pool(h, batch3d, linw, linb):
    n, d = h.shape
    c = linw.shape[1]
    grid = n // ROW_BLOCK
    return pl.pallas_call(
        _pool_body,
        grid=(grid,),
        in_specs=[
            pl.BlockSpec((ROW_BLOCK, d), lambda i: (i, 0)),
            pl.BlockSpec((1, 1, ROW_BLOCK), lambda i: (i, 0, 0)),
            pl.BlockSpec((d, c), lambda i: (0, 0)),
            pl.BlockSpec((c,), lambda i: (0,)),
        ],
        out_specs=pl.BlockSpec((NUM_GRAPHS, c), lambda i: (0, 0)),
        out_shape=jax.ShapeDtypeStruct((NUM_GRAPHS, c), jnp.float32),
        scratch_shapes=[
            pltpu.VMEM((NUM_GRAPHS, d), jnp.float32),
            pltpu.VMEM((1, NUM_GRAPHS), jnp.float32),
        ],
    )(h, batch3d, linw, linb)


NB = 8
NCK = 6250
PADC = 6400
DUM = NCK
ET = 25600
CHK = 1600
CAPROWS = 6400
SCP = NB // NSC


def _iota16():
    return lax.iota(jnp.int32, 16)


def _extract(vec16, i):
    return jnp.sum(jnp.where(_iota16() == i, vec16, 0))


def _sc_params():
    return pltpu.CompilerParams(use_tc_tiling_on_sc=False, needs_layout_passes=False)


def _sc_count(dst_pad):
    NCH = ET // CHK

    def body(dst_hbm, out_hbm, chunk0, chunk1, row_v, sem):
        w = _wid()
        bufs = [chunk0, chunk1]
        pend = pltpu.async_copy(dst_hbm.at[pl.ds(w * ET, CHK)], chunk0, sem)
        cnts = tuple(jnp.zeros((16,), jnp.int32) for _ in range(NB))
        for ci in range(NCH):
            pend.wait()
            if ci + 1 < NCH:
                pend = pltpu.async_copy(
                    dst_hbm.at[pl.ds(w * ET + (ci + 1) * CHK, CHK)],
                    bufs[(ci + 1) % 2], sem)
            buf = bufs[ci % 2]

            def vec_body(vi, cnts):
                d = buf[pl.ds(vi * 16, 16)]
                bkt = d // NCK
                return tuple(cnts[b] + (bkt == b).astype(jnp.int32)
                             for b in range(NB))

            cnts = lax.fori_loop(0, CHK // 16, vec_body, cnts)
        row = jnp.zeros((16,), jnp.int32)
        for b in range(NB):
            row = jnp.where(_iota16() == b, jnp.sum(cnts[b]), row)
        row_v[...] = row
        pltpu.sync_copy(row_v, out_hbm.at[w])

    f = pl.kernel(
        body,
        out_type=jax.ShapeDtypeStruct((NW, 16), jnp.int32),
        mesh=_mesh(),
        scratch_types=[
            pltpu.VMEM((CHK,), jnp.int32),
            pltpu.VMEM((CHK,), jnp.int32),
            pltpu.VMEM((16,), jnp.int32),
            pltpu.SemaphoreType.DMA,
        ],
        compiler_params=_sc_params(),
    )
    return f(dst_pad)


def _sc_partition(src_pad, dst_pad, typ_pad, slot_off, loc_off, cnt_wb, dummy_meta):
    STG = ET + NB * 16

    def body(src_hbm, dst_hbm, typ_hbm, slot_hbm, loc_hbm, cnt_hbm, dmy_hbm,
             srcp_hbm, combp_hbm,
             srcv0, dstv0, typv0, srcv1, dstv1, typv1,
             sstage, cstage, m16, dzero, ddum, semc, sem):
        w = _wid()
        srcb = [srcv0, srcv1]
        dstb = [dstv0, dstv1]
        typb = [typv0, typv1]

        pltpu.sync_copy(slot_hbm.at[w], m16)
        slot = m16[...]
        pltpu.sync_copy(loc_hbm.at[w], m16)
        loc = m16[...]
        pltpu.sync_copy(cnt_hbm.at[w], m16)
        cnt = m16[...]

        for i in range(8):
            dzero[pl.ds(i * 16, 16)] = jnp.zeros((16,), jnp.int32)
            ddum[pl.ds(i * 16, 16)] = jnp.full((16,), DUM, jnp.int32)

        @pl.when(w < NB)
        def _dummies():
            pltpu.sync_copy(dmy_hbm.at[0], m16)
            offa = pl.multiple_of(_extract(m16[...], w), 8)
            pltpu.sync_copy(dmy_hbm.at[1], m16)
            offb = pl.multiple_of(_extract(m16[...], w), 8)
            pltpu.sync_copy(dzero, srcp_hbm.at[pl.ds(offa, 128)])
            pltpu.sync_copy(dzero, srcp_hbm.at[pl.ds(offb, 128)])
            pltpu.sync_copy(ddum, combp_hbm.at[pl.ds(offa, 128)])
            pltpu.sync_copy(ddum, combp_hbm.at[pl.ds(offb, 128)])

        cur = tuple(_extract(loc, b) for b in range(NB))
        NCH = ET // CHK

        def load(ci, which):
            base = w * ET + ci * CHK
            return [pltpu.async_copy(src_hbm.at[pl.ds(base, CHK)],
                                     srcb[which], semc),
                    pltpu.async_copy(dst_hbm.at[pl.ds(base, CHK)],
                                     dstb[which], semc),
                    pltpu.async_copy(typ_hbm.at[pl.ds(base, CHK)],
                                     typb[which], semc)]

        pend = load(0, 0)
        for ci in range(NCH):
            for h in pend:
                h.wait()
            if ci + 1 < NCH:
                pend = load(ci + 1, (ci + 1) % 2)
            srcv, dstv, typv = srcb[ci % 2], dstb[ci % 2], typb[ci % 2]

            def vec_body(vi, cur, srcv=srcv, dstv=dstv, typv=typv):
                s = srcv[pl.ds(vi * 16, 16)]
                d = dstv[pl.ds(vi * 16, 16)]
                t = typv[pl.ds(vi * 16, 16)]
                bkt = d // NCK
                cb = t * PADC + (d - bkt * NCK)
                out = []
                for b in range(NB):
                    m = bkt == b
                    mi = m.astype(jnp.int32)
                    off = cur[b] + plsc.cumsum(mi) - 1
                    plsc.store_scatter(sstage, [off], s, mask=m)
                    plsc.store_scatter(cstage, [off], cb, mask=m)
                    out.append(cur[b] + jnp.sum(mi))
                return tuple(out)

            cur = lax.fori_loop(0, CHK // 16, vec_body, cur)

        for b in range(NB):
            nb_cnt = _extract(cnt, b)
            pc = (nb_cnt + 7) & ~jnp.int32(7)
            toff = cur[b] + _iota16()
            tm = _iota16() < (pc - nb_cnt)
            plsc.store_scatter(sstage, [toff], jnp.zeros((16,), jnp.int32),
                               mask=tm)
            plsc.store_scatter(cstage, [toff], jnp.full((16,), DUM, jnp.int32),
                               mask=tm)
            lo = _extract(loc, b)
            go = _extract(slot, b)

            def drain(step, j0):
                def cond(j):
                    return j + step <= pc

                def dbody(j):
                    lj = pl.multiple_of(lo + j, 8)
                    gj = pl.multiple_of(go + j, 8)
                    pltpu.sync_copy(sstage.at[pl.ds(lj, step)],
                                    srcp_hbm.at[pl.ds(gj, step)])
                    pltpu.sync_copy(cstage.at[pl.ds(lj, step)],
                                    combp_hbm.at[pl.ds(gj, step)])
                    return j + step

                return lax.while_loop(cond, dbody, j0)

            j = drain(512, jnp.int32(0))
            j = drain(64, j)
            drain(8, j)

    f = pl.kernel(
        body,
        out_type=(jax.ShapeDtypeStruct((CAPROWS * 128,), jnp.int32),
                  jax.ShapeDtypeStruct((CAPROWS * 128,), jnp.int32)),
        mesh=_mesh(),
        scratch_types=[
            pltpu.VMEM((CHK,), jnp.int32),
            pltpu.VMEM((CHK,), jnp.int32),
            pltpu.VMEM((CHK,), jnp.int32),
            pltpu.VMEM((CHK,), jnp.int32),
            pltpu.VMEM((CHK,), jnp.int32),
            pltpu.VMEM((CHK,), jnp.int32),
            pltpu.VMEM((STG,), jnp.int32),
            pltpu.VMEM((STG,), jnp.int32),
            pltpu.VMEM((16,), jnp.int32),
            pltpu.VMEM((128,), jnp.int32),
            pltpu.VMEM((128,), jnp.int32),
            pltpu.SemaphoreType.DMA,
            pltpu.SemaphoreType.DMA,
        ],
        compiler_params=_sc_params(),
    )
    return f(src_pad, dst_pad, typ_pad, slot_off, loc_off, cnt_wb, dummy_meta)


def _sc_hist(comb_part, rowmeta):
    GD = 4

    def body(comb_hbm, meta_hbm, out_hbm, m16, c0, c1, c2, c3, hist, sem):
        w = _wid()
        b = w // 4
        cb = [c0, c1, c2, c3]
        pltpu.sync_copy(meta_hbm.at[0], m16)
        meta = m16[...]
        start = _extract(meta, b)
        end = _extract(meta, b + 1)

        def zbody(i, _):
            hist[pl.ds(i * 16, 16)] = jnp.zeros((16,), jnp.float32)
            return ()

        lax.fori_loop(0, NUM_REL * PADC // 16, zbody, ())

        def scat(buf):
            def vbody(vi, _):
                c = buf[pl.ds(vi * 16, 16)]
                plsc.addupdate_scatter(hist, [c], jnp.ones((16,), jnp.float32))
                return ()

            lax.fori_loop(0, 8, vbody, ())

        def gcond(k):
            return k + 4 * (GD - 1) < end

        def gbody(k):
            hs = [pltpu.async_copy(
                comb_hbm.at[pl.ds((k + 4 * s) * 128, 128)], cb[s], sem)
                for s in range(GD)]
            for s in range(GD):
                hs[s].wait()
                scat(cb[s])
            return k + 4 * GD

        k = lax.while_loop(gcond, gbody, start + (w % 4))

        def cond(k):
            return k < end

        def kbody(k):
            pltpu.sync_copy(comb_hbm.at[pl.ds(k * 128, 128)], cb[0])
            scat(cb[0])
            return k + 4

        lax.while_loop(cond, kbody, k)
        pltpu.sync_copy(hist, out_hbm.at[w])

    f = pl.kernel(
        body,
        out_type=jax.ShapeDtypeStruct((NW, NUM_REL * PADC), jnp.float32),
        mesh=_mesh(),
        scratch_types=[
            pltpu.VMEM((16,), jnp.int32),
            pltpu.VMEM((128,), jnp.int32),
            pltpu.VMEM((128,), jnp.int32),
            pltpu.VMEM((128,), jnp.int32),
            pltpu.VMEM((128,), jnp.int32),
            pltpu.VMEM((NUM_REL * PADC,), jnp.float32),
            pltpu.SemaphoreType.DMA,
        ],
        compiler_params=_sc_params(),
    )
    return f(comb_part, rowmeta)


def _sc_aggregate(h, src_part, comb_part, rowmeta, zeros_rows, n):
    D = 3
    STRIDE = 16 * 2 * D

    def body(h_hbm, srcp_hbm, combp_hbm, meta_hbm, z_hbm, agg_hbm,
             m16, *rest):
        acc_sh, semia, semib, semg, sema = rest[6 * D:]
        sidxA = [rest[6 * s] for s in range(D)]
        cidxA = [rest[6 * s + 1] for s in range(D)]
        rowsA = [rest[6 * s + 2] for s in range(D)]
        sidxB = [rest[6 * s + 3] for s in range(D)]
        cidxB = [rest[6 * s + 4] for s in range(D)]
        rowsB = [rest[6 * s + 5] for s in range(D)]
        c = lax.axis_index("c")
        tid = lax.axis_index("s")
        pltpu.sync_copy(meta_hbm.at[0], m16)
        meta = m16[...]

        for cpass in range(SCP):
            b = c * SCP + cpass
            start = _extract(meta, b)
            end = _extract(meta, b + 1)

            def cl(kk):
                return jnp.where(kk < end, kk, start)

            def issue_src(sidx, kk, semI):
                for s in range(D):
                    base = cl(kk + 16 * s) * 128
                    pltpu.async_copy(srcp_hbm.at[pl.ds(base, 128)],
                                     sidx[s], semI)

            def issue_comb(cidx, kk, semI):
                for s in range(D):
                    base = cl(kk + 16 * s) * 128
                    pltpu.async_copy(combp_hbm.at[pl.ds(base, 128)],
                                     cidx[s], semI)

            def wait_idx(sidx, cidx, kk, semI):
                for s in range(D):
                    base = cl(kk + 16 * s) * 128
                    pltpu.make_async_copy(srcp_hbm.at[pl.ds(base, 128)],
                                          sidx[s], semI).wait()
                    pltpu.make_async_copy(combp_hbm.at[pl.ds(base, 128)],
                                          cidx[s], semI).wait()

            pltpu.sync_copy(z_hbm, rowsA[0])
            z0 = tid * (NUM_REL * PADC // NTILE)
            zh = [pltpu.async_copy(
                rowsA[0], acc_sh.at[pl.ds(z0 + zi * 128, 128), :], semg)
                for zi in range(9)]
            zh.append(pltpu.async_copy(
                rowsA[0].at[pl.ds(0, 48), :],
                acc_sh.at[pl.ds(z0 + 9 * 128, 48), :], semg))
            for h in zh:
                h.wait()
            plsc.subcore_barrier()

            k0 = start + tid
            issue_src(sidxA, k0, semia)
            issue_comb(cidxA, k0, semia)
            issue_src(sidxB, k0 + 16 * D, semib)
            issue_comb(cidxB, k0 + 16 * D, semib)

            def half(sidx, cidx, rows, kk, semI, knext):
                wait_idx(sidx, cidx, kk, semI)
                hg = [pltpu.async_copy(h_hbm.at[sidx[s]], rows[s], semg)
                      for s in range(D)]
                ha = []
                for s in range(D):
                    hg[s].wait()
                    ha.append(pltpu.async_copy(rows[s], acc_sh.at[cidx[s]],
                                               sema, add=True))
                issue_src(sidx, knext, semI)
                return ha

            def condg(k):
                return k + 16 * (2 * D - 1) < end

            def group(k):
                ha = half(sidxA, cidxA, rowsA, k, semia, k + STRIDE)
                hb = half(sidxB, cidxB, rowsB, k + 16 * D, semib,
                          k + STRIDE + 16 * D)
                for h in ha + hb:
                    h.wait()
                issue_comb(cidxA, k + STRIDE, semia)
                issue_comb(cidxB, k + STRIDE + 16 * D, semib)
                return k + STRIDE

            k = lax.while_loop(condg, group, k0)

            for (sidx, cidx, rows, kk, semI) in (
                    (sidxA, cidxA, rowsA, k, semia),
                    (sidxB, cidxB, rowsB, k + 16 * D, semib)):
                wait_idx(sidx, cidx, kk, semI)
                for s in range(D):
                    def _do(sidx=sidx, cidx=cidx, rows=rows, s=s):
                        pltpu.async_copy(h_hbm.at[sidx[s]], rows[s],
                                         semg).wait()
                        pltpu.async_copy(rows[s], acc_sh.at[cidx[s]],
                                         sema, add=True).wait()
                    pl.when(kk + 16 * s < end)(_do)

            plsc.subcore_barrier()

            def dcond(m):
                return m < NCK // 125

            def dbody(m):
                hd = [pltpu.async_copy(
                    acc_sh.at[pl.ds(r * PADC + m * 125, 125), :],
                    agg_hbm.at[r, pl.ds(b * NCK + m * 125, 125), :], semg)
                    for r in range(NUM_REL)]
                for h in hd:
                    h.wait()
                return m + 16

            lax.while_loop(dcond, dbody, tid)
            plsc.subcore_barrier()

    f = pl.kernel(
        body,
        out_type=jax.ShapeDtypeStruct((NUM_REL, n, 64), jnp.float32),
        mesh=_mesh(),
        scratch_types=[
            pltpu.VMEM((16,), jnp.int32),
        ] + [
            t
            for _ in range(D)
            for t in (pltpu.VMEM((128,), jnp.int32),
                      pltpu.VMEM((128,), jnp.int32),
                      pltpu.VMEM((128, 64), jnp.float32),
                      pltpu.VMEM((128,), jnp.int32),
                      pltpu.VMEM((128,), jnp.int32),
                      pltpu.VMEM((128, 64), jnp.float32))
        ] + [
            pltpu.VMEM_SHARED((NUM_REL * PADC, 64), jnp.float32),
            pltpu.SemaphoreType.DMA,
            pltpu.SemaphoreType.DMA,
            pltpu.SemaphoreType.DMA,
            pltpu.SemaphoreType.DMA,
        ],
        compiler_params=_sc_params(),
    )
    return f(h, src_part, comb_part, rowmeta, zeros_rows)


def _partition_glue(cnt_wb):
    cnt = cnt_wb[:, :NB]
    pc = (cnt + 7) & ~jnp.int32(7)
    u = jnp.sum(pc, axis=0)
    rows = (u + 255) // 128
    start_row = jnp.concatenate([jnp.zeros((1,), jnp.int32),
                                 jnp.cumsum(rows)]).astype(jnp.int32)
    start_ent = start_row[:NB] * 128
    slot_off = start_ent[None, :] + (jnp.cumsum(pc, axis=0) - pc)
    loc_off = jnp.cumsum(pc, axis=1) - pc
    dummy_a = start_ent + u
    dummy_b = start_row[1:] * 128 - 128

    def pad16(a):
        return jnp.concatenate(
            [a.astype(jnp.int32),
             jnp.zeros(a.shape[:-1] + (16 - a.shape[-1],), jnp.int32)], -1)

    rowmeta = pad16(start_row[None, :NB + 1])
    dummy_meta = jnp.stack([pad16(dummy_a[None, :])[0],
                            pad16(dummy_b[None, :])[0]])
    return pad16(slot_off), pad16(loc_off), rowmeta, dummy_meta


def kernel(x, edge_index, edge_type, batch, embed, W1, Wroot1, b1, W2, Wroot2, b2, linW, linb):
    n = x.shape[0]
    e = edge_index.shape[1]
    src, dst = edge_index[0], edge_index[1]

    npad = ((n + 128 * NW - 1) // (128 * NW)) * (128 * NW)
    x_pad = jnp.concatenate([x.astype(jnp.int32), jnp.zeros((npad - n,), jnp.int32)])
    h0 = _sc_embed_gather(x_pad, embed)

    epad = NW * ET
    src_p = jnp.concatenate([src.astype(jnp.int32), jnp.zeros((epad - e,), jnp.int32)])
    dst_p = jnp.concatenate([dst.astype(jnp.int32), jnp.full((epad - e,), n, jnp.int32)])
    typ_p = jnp.concatenate([edge_type.astype(jnp.int32), jnp.zeros((epad - e,), jnp.int32)])

    cnt_wb = _sc_count(dst_p)
    slot_off, loc_off, rowmeta, dummy_meta = _partition_glue(cnt_wb)
    src_part, comb_part = _sc_partition(src_p, dst_p, typ_p, slot_off, loc_off,
                                        cnt_wb, dummy_meta)
    hist = _sc_hist(comb_part, rowmeta)
    cnt = (hist.reshape(NB, 4, NUM_REL, PADC).sum(axis=1)[:, :, :NCK]
           .transpose(1, 0, 2).reshape(NUM_REL, n))
    icnt = (1.0 / jnp.maximum(cnt, 1.0)).T
    batch3d = batch.reshape(n // ROW_BLOCK, 1, ROW_BLOCK)
    zrows = jnp.zeros((128, 64), jnp.float32)

    agg1 = _sc_aggregate(h0, src_part, comb_part, rowmeta, zrows, n)
    h1 = _tc_dense(h0, agg1, icnt, Wroot1, b1, W1, n=n)
    agg2 = _sc_aggregate(h1, src_part, comb_part, rowmeta, zrows, n)
    h2 = _tc_dense(h1, agg2, icnt, Wroot2, b2, W2)
    return _tc_pool(h2, batch3d, linW, linb)


# speedup vs baseline: 14.5881x; 1.0007x over previous
import jax
import jax.numpy as jnp
from jax import lax
from jax.experimental import pallas as pl
from jax.experimental.pallas import tpu as pltpu
from jax.experimental.pallas import tpu_sc as plsc

NUM_REL = 3
NUM_GRAPHS = 64
ROW_BLOCK = 2000

NSC = 2
NTILE = 16
NW = NSC * NTILE


def _mesh():
    return plsc.VectorSubcoreMesh(core_axis_name="c", subcore_axis_name="s")


def _wid():
    return lax.axis_index("s") * NSC + lax.axis_index("c")


def _sc_embed_gather(x_pad, embed):
    npad, d = x_pad.shape[0], embed.shape[1]
    nch = npad // 128
    per_w = nch // NW

    def body(x_hbm, table_hbm, out_hbm, idx0, idx1, rows0, rows1,
             semi, semg, semw):
        w = _wid()
        idx = [idx0, idx1]
        rows = [rows0, rows1]
        pend_w = [None, None]
        pend_i = pltpu.async_copy(x_hbm.at[pl.ds(w * per_w * 128, 128)],
                                  idx0, semi)
        for j in range(per_w):
            k = w * per_w + j
            pend_i.wait()
            if j + 1 < per_w:
                pend_i = pltpu.async_copy(
                    x_hbm.at[pl.ds((k + 1) * 128, 128)], idx[(j + 1) % 2],
                    semi)
            if pend_w[j % 2] is not None:
                pend_w[j % 2].wait()
            pltpu.async_copy(table_hbm.at[idx[j % 2]], rows[j % 2],
                             semg).wait()
            pend_w[j % 2] = pltpu.async_copy(
                rows[j % 2], out_hbm.at[pl.ds(k * 128, 128), :], semw)
        for p in pend_w:
            if p is not None:
                p.wait()

    f = pl.kernel(
        body,
        out_type=jax.ShapeDtypeStruct((npad, d), jnp.float32),
        mesh=_mesh(),
        scratch_types=[
            pltpu.VMEM((128,), jnp.int32),
            pltpu.VMEM((128,), jnp.int32),
            pltpu.VMEM((128, d), jnp.float32),
            pltpu.VMEM((128, d), jnp.float32),
            pltpu.SemaphoreType.DMA,
            pltpu.SemaphoreType.DMA,
            pltpu.SemaphoreType.DMA,
        ],
        compiler_params=pltpu.CompilerParams(use_tc_tiling_on_sc=False, needs_layout_passes=False),
    )
    return f(x_pad, embed)


def _dense_body(h_ref, agg_ref, icnt_ref, wroot_ref, b_ref, w_ref, out_ref):
    h = h_ref[...]
    acc = jnp.dot(h, wroot_ref[...], preferred_element_type=jnp.float32)
    acc = acc + b_ref[...][None, :]
    for r in range(NUM_REL):
        m = agg_ref[r] * icnt_ref[:, r][:, None]
        acc = acc + jnp.dot(m, w_ref[r], preferred_element_type=jnp.float32)
    out_ref[...] = jnp.maximum(acc, 0.0)


def _tc_dense(h, agg, icnt, wroot, b, w, n=None):
    d = h.shape[1]
    if n is None:
        n = h.shape[0]
    hh = w.shape[2]
    grid = n // ROW_BLOCK
    return pl.pallas_call(
        _dense_body,
        grid=(grid,),
        in_specs=[
            pl.BlockSpec((ROW_BLOCK, d), lambda i: (i, 0)),
            pl.BlockSpec((NUM_REL, ROW_BLOCK, d), lambda i: (0, i, 0)),
            pl.BlockSpec((ROW_BLOCK, NUM_REL), lambda i: (i, 0)),
            pl.BlockSpec((d, hh), lambda i: (0, 0)),
            pl.BlockSpec((hh,), lambda i: (0,)),
            pl.BlockSpec((NUM_REL, d, hh), lambda i: (0, 0, 0)),
        ],
        out_specs=pl.BlockSpec((ROW_BLOCK, hh), lambda i: (i, 0)),
        out_shape=jax.ShapeDtypeStruct((n, hh), jnp.float32),
    )(h, agg, icnt, wroot, b, w)


def _pool_body(h_ref, batch_ref, linw_ref, linb_ref, out_ref, acc_ref, cnt_ref):
    i = pl.program_id(0)

    @pl.when(i == 0)
    def _init():
        acc_ref[...] = jnp.zeros_like(acc_ref)
        cnt_ref[...] = jnp.zeros_like(cnt_ref)

    bvec = batch_ref[0, 0, :]
    iota = lax.broadcasted_iota(jnp.int32, (ROW_BLOCK, NUM_GRAPHS), 1)
    onehot = (bvec[:, None] == iota).astype(jnp.float32)
    acc_ref[...] += lax.dot_general(
        onehot, h_ref[...], (((0,), (0,)), ((), ())),
        preferred_element_type=jnp.float32)
    cnt_ref[...] += jnp.sum(onehot, axis=0, keepdims=True)

    @pl.when(i == pl.num_programs(0) - 1)
    def _fin():
        pooled = acc_ref[...] / jnp.maximum(cnt_ref[...], 1.0).T
        out_ref[...] = jnp.dot(pooled, linw_ref[...],
                               preferred_element_type=jnp.float32) + linb_ref[...][None, :]


def _tc_pool(h, batch3d, linw, linb):
    n, d = h.shape
    c = linw.shape[1]
    grid = n // ROW_BLOCK
    return pl.pallas_call(
        _pool_body,
        grid=(grid,),
        in_specs=[
            pl.BlockSpec((ROW_BLOCK, d), lambda i: (i, 0)),
            pl.BlockSpec((1, 1, ROW_BLOCK), lambda i: (i, 0, 0)),
            pl.BlockSpec((d, c), lambda i: (0, 0)),
            pl.BlockSpec((c,), lambda i: (0,)),
        ],
        out_specs=pl.BlockSpec((NUM_GRAPHS, c), lambda i: (0, 0)),
        out_shape=jax.ShapeDtypeStruct((NUM_GRAPHS, c), jnp.float32),
        scratch_shapes=[
            pltpu.VMEM((NUM_GRAPHS, d), jnp.float32),
            pltpu.VMEM((1, NUM_GRAPHS), jnp.float32),
        ],
    )(h, batch3d, linw, linb)


NB = 8
NCK = 6250
PADC = 6400
DUM = NCK
ET = 25600
CHK = 1600
CAPROWS = 6400
SCP = NB // NSC


def _iota16():
    return lax.iota(jnp.int32, 16)


def _extract(vec16, i):
    return jnp.sum(jnp.where(_iota16() == i, vec16, 0))


def _sc_params():
    return pltpu.CompilerParams(use_tc_tiling_on_sc=False, needs_layout_passes=False)


def _sc_count(dst_pad):
    NCH = ET // CHK

    def body(dst_hbm, out_hbm, chunk0, chunk1, row_v, sem):
        w = _wid()
        bufs = [chunk0, chunk1]
        pend = pltpu.async_copy(dst_hbm.at[pl.ds(w * ET, CHK)], chunk0, sem)
        cnts = tuple(jnp.zeros((16,), jnp.int32) for _ in range(NB))
        for ci in range(NCH):
            pend.wait()
            if ci + 1 < NCH:
                pend = pltpu.async_copy(
                    dst_hbm.at[pl.ds(w * ET + (ci + 1) * CHK, CHK)],
                    bufs[(ci + 1) % 2], sem)
            buf = bufs[ci % 2]

            def vec_body(vi, cnts):
                d = buf[pl.ds(vi * 16, 16)]
                bkt = d // NCK
                return tuple(cnts[b] + (bkt == b).astype(jnp.int32)
                             for b in range(NB))

            cnts = lax.fori_loop(0, CHK // 16, vec_body, cnts)
        row = jnp.zeros((16,), jnp.int32)
        for b in range(NB):
            row = jnp.where(_iota16() == b, jnp.sum(cnts[b]), row)
        row_v[...] = row
        pltpu.sync_copy(row_v, out_hbm.at[w])

    f = pl.kernel(
        body,
        out_type=jax.ShapeDtypeStruct((NW, 16), jnp.int32),
        mesh=_mesh(),
        scratch_types=[
            pltpu.VMEM((CHK,), jnp.int32),
            pltpu.VMEM((CHK,), jnp.int32),
            pltpu.VMEM((16,), jnp.int32),
            pltpu.SemaphoreType.DMA,
        ],
        compiler_params=_sc_params(),
    )
    return f(dst_pad)


def _sc_partition(src_pad, dst_pad, typ_pad, slot_off, loc_off, cnt_wb, dummy_meta):
    STG = ET + NB * 16

    def body(src_hbm, dst_hbm, typ_hbm, slot_hbm, loc_hbm, cnt_hbm, dmy_hbm,
             srcp_hbm, combp_hbm,
             srcv0, dstv0, typv0, srcv1, dstv1, typv1,
             sstage, cstage, m16, dzero, ddum, semc, sem):
        w = _wid()
        srcb = [srcv0, srcv1]
        dstb = [dstv0, dstv1]
        typb = [typv0, typv1]

        pltpu.sync_copy(slot_hbm.at[w], m16)
        slot = m16[...]
        pltpu.sync_copy(loc_hbm.at[w], m16)
        loc = m16[...]
        pltpu.sync_copy(cnt_hbm.at[w], m16)
        cnt = m16[...]

        for i in range(8):
            dzero[pl.ds(i * 16, 16)] = jnp.zeros((16,), jnp.int32)
            ddum[pl.ds(i * 16, 16)] = jnp.full((16,), DUM, jnp.int32)

        @pl.when(w < NB)
        def _dummies():
            pltpu.sync_copy(dmy_hbm.at[0], m16)
            offa = pl.multiple_of(_extract(m16[...], w), 8)
            pltpu.sync_copy(dmy_hbm.at[1], m16)
            offb = pl.multiple_of(_extract(m16[...], w), 8)
            pltpu.sync_copy(dzero, srcp_hbm.at[pl.ds(offa, 128)])
            pltpu.sync_copy(dzero, srcp_hbm.at[pl.ds(offb, 128)])
            pltpu.sync_copy(ddum, combp_hbm.at[pl.ds(offa, 128)])
            pltpu.sync_copy(ddum, combp_hbm.at[pl.ds(offb, 128)])

        cur = tuple(_extract(loc, b) for b in range(NB))
        NCH = ET // CHK

        def load(ci, which):
            base = w * ET + ci * CHK
            return [pltpu.async_copy(src_hbm.at[pl.ds(base, CHK)],
                                     srcb[which], semc),
                    pltpu.async_copy(dst_hbm.at[pl.ds(base, CHK)],
                                     dstb[which], semc),
                    pltpu.async_copy(typ_hbm.at[pl.ds(base, CHK)],
                                     typb[which], semc)]

        pend = load(0, 0)
        for ci in range(NCH):
            for h in pend:
                h.wait()
            if ci + 1 < NCH:
                pend = load(ci + 1, (ci + 1) % 2)
            srcv, dstv, typv = srcb[ci % 2], dstb[ci % 2], typb[ci % 2]

            def vec_body(vi, cur, srcv=srcv, dstv=dstv, typv=typv):
                s = srcv[pl.ds(vi * 16, 16)]
                d = dstv[pl.ds(vi * 16, 16)]
                t = typv[pl.ds(vi * 16, 16)]
                bkt = d // NCK
                cb = t * PADC + (d - bkt * NCK)
                out = []
                for b in range(NB):
                    m = bkt == b
                    mi = m.astype(jnp.int32)
                    off = cur[b] + plsc.cumsum(mi) - 1
                    plsc.store_scatter(sstage, [off], s, mask=m)
                    plsc.store_scatter(cstage, [off], cb, mask=m)
                    out.append(cur[b] + jnp.sum(mi))
                return tuple(out)

            cur = lax.fori_loop(0, CHK // 16, vec_body, cur)

        for b in range(NB):
            nb_cnt = _extract(cnt, b)
            pc = (nb_cnt + 7) & ~jnp.int32(7)
            toff = cur[b] + _iota16()
            tm = _iota16() < (pc - nb_cnt)
            plsc.store_scatter(sstage, [toff], jnp.zeros((16,), jnp.int32),
                               mask=tm)
            plsc.store_scatter(cstage, [toff], jnp.full((16,), DUM, jnp.int32),
                               mask=tm)
            lo = _extract(loc, b)
            go = _extract(slot, b)

            def drain(step, j0):
                def cond(j):
                    return j + step <= pc

                def dbody(j):
                    lj = pl.multiple_of(lo + j, 8)
                    gj = pl.multiple_of(go + j, 8)
                    pltpu.sync_copy(sstage.at[pl.ds(lj, step)],
                                    srcp_hbm.at[pl.ds(gj, step)])
                    pltpu.sync_copy(cstage.at[pl.ds(lj, step)],
                                    combp_hbm.at[pl.ds(gj, step)])
                    return j + step

                return lax.while_loop(cond, dbody, j0)

            j = drain(512, jnp.int32(0))
            j = drain(64, j)
            drain(8, j)

    f = pl.kernel(
        body,
        out_type=(jax.ShapeDtypeStruct((CAPROWS * 128,), jnp.int32),
                  jax.ShapeDtypeStruct((CAPROWS * 128,), jnp.int32)),
        mesh=_mesh(),
        scratch_types=[
            pltpu.VMEM((CHK,), jnp.int32),
            pltpu.VMEM((CHK,), jnp.int32),
            pltpu.VMEM((CHK,), jnp.int32),
            pltpu.VMEM((CHK,), jnp.int32),
            pltpu.VMEM((CHK,), jnp.int32),
            pltpu.VMEM((CHK,), jnp.int32),
            pltpu.VMEM((STG,), jnp.int32),
            pltpu.VMEM((STG,), jnp.int32),
            pltpu.VMEM((16,), jnp.int32),
            pltpu.VMEM((128,), jnp.int32),
            pltpu.VMEM((128,), jnp.int32),
            pltpu.SemaphoreType.DMA,
            pltpu.SemaphoreType.DMA,
        ],
        compiler_params=_sc_params(),
    )
    return f(src_pad, dst_pad, typ_pad, slot_off, loc_off, cnt_wb, dummy_meta)


def _sc_hist(comb_part, rowmeta):
    GD = 4

    def body(comb_hbm, meta_hbm, out_hbm, m16, c0, c1, c2, c3, hist, sem):
        w = _wid()
        b = w // 4
        cb = [c0, c1, c2, c3]
        pltpu.sync_copy(meta_hbm.at[0], m16)
        meta = m16[...]
        start = _extract(meta, b)
        end = _extract(meta, b + 1)

        def zbody(i, _):
            hist[pl.ds(i * 16, 16)] = jnp.zeros((16,), jnp.float32)
            return ()

        lax.fori_loop(0, NUM_REL * PADC // 16, zbody, ())

        def scat(buf):
            def vbody(vi, _):
                c = buf[pl.ds(vi * 16, 16)]
                plsc.addupdate_scatter(hist, [c], jnp.ones((16,), jnp.float32))
                return ()

            lax.fori_loop(0, 8, vbody, ())

        def gcond(k):
            return k + 4 * (GD - 1) < end

        def gbody(k):
            hs = [pltpu.async_copy(
                comb_hbm.at[pl.ds((k + 4 * s) * 128, 128)], cb[s], sem)
                for s in range(GD)]
            for s in range(GD):
                hs[s].wait()
                scat(cb[s])
            return k + 4 * GD

        k = lax.while_loop(gcond, gbody, start + (w % 4))

        def cond(k):
            return k < end

        def kbody(k):
            pltpu.sync_copy(comb_hbm.at[pl.ds(k * 128, 128)], cb[0])
            scat(cb[0])
            return k + 4

        lax.while_loop(cond, kbody, k)
        pltpu.sync_copy(hist, out_hbm.at[w])

    f = pl.kernel(
        body,
        out_type=jax.ShapeDtypeStruct((NW, NUM_REL * PADC), jnp.float32),
        mesh=_mesh(),
        scratch_types=[
            pltpu.VMEM((16,), jnp.int32),
            pltpu.VMEM((128,), jnp.int32),
            pltpu.VMEM((128,), jnp.int32),
            pltpu.VMEM((128,), jnp.int32),
            pltpu.VMEM((128,), jnp.int32),
            pltpu.VMEM((NUM_REL * PADC,), jnp.float32),
            pltpu.SemaphoreType.DMA,
        ],
        compiler_params=_sc_params(),
    )
    return f(comb_part, rowmeta)


def _sc_aggregate(h, src_part, comb_part, rowmeta, zeros_rows, n):
    D = 3
    STRIDE = 16 * 2 * D

    def body(h_hbm, srcp_hbm, combp_hbm, meta_hbm, z_hbm, agg_hbm,
             m16, *rest):
        acc_sh, semia, semib, semg, sema = rest[6 * D:]
        sidxA = [rest[6 * s] for s in range(D)]
        cidxA = [rest[6 * s + 1] for s in range(D)]
        rowsA = [rest[6 * s + 2] for s in range(D)]
        sidxB = [rest[6 * s + 3] for s in range(D)]
        cidxB = [rest[6 * s + 4] for s in range(D)]
        rowsB = [rest[6 * s + 5] for s in range(D)]
        c = lax.axis_index("c")
        tid = lax.axis_index("s")
        pltpu.sync_copy(meta_hbm.at[0], m16)
        meta = m16[...]

        for cpass in range(SCP):
            b = c * SCP + cpass
            start = _extract(meta, b)
            end = _extract(meta, b + 1)

            def cl(kk):
                return jnp.where(kk < end, kk, start)

            def issue_src(sidx, kk, semI):
                for s in range(D):
                    base = cl(kk + 16 * s) * 128
                    pltpu.async_copy(srcp_hbm.at[pl.ds(base, 128)],
                                     sidx[s], semI)

            def issue_comb(cidx, kk, semI):
                for s in range(D):
                    base = cl(kk + 16 * s) * 128
                    pltpu.async_copy(combp_hbm.at[pl.ds(base, 128)],
                                     cidx[s], semI)

            def wait_idx(sidx, cidx, kk, semI):
                for s in range(D):
                    base = cl(kk + 16 * s) * 128
                    pltpu.make_async_copy(srcp_hbm.at[pl.ds(base, 128)],
                                          sidx[s], semI).wait()
                    pltpu.make_async_copy(combp_hbm.at[pl.ds(base, 128)],
                                          cidx[s], semI).wait()

            pltpu.sync_copy(z_hbm, rowsA[0])
            z0 = tid * (NUM_REL * PADC // NTILE)
            zh = [pltpu.async_copy(
                rowsA[0], acc_sh.at[pl.ds(z0 + zi * 128, 128), :], semg)
                for zi in range(9)]
            zh.append(pltpu.async_copy(
                rowsA[0].at[pl.ds(0, 48), :],
                acc_sh.at[pl.ds(z0 + 9 * 128, 48), :], semg))
            for h in zh:
                h.wait()
            plsc.subcore_barrier()

            k0 = start + tid
            issue_src(sidxA, k0, semia)
            issue_comb(cidxA, k0, semia)
            issue_src(sidxB, k0 + 16 * D, semib)
            issue_comb(cidxB, k0 + 16 * D, semib)

            def half(sidx, cidx, rows, kk, semI, knext):
                wait_idx(sidx, cidx, kk, semI)
                hg = [pltpu.async_copy(h_hbm.at[sidx[s]], rows[s], semg)
                      for s in range(D)]
                ha = []
                for s in range(D):
                    hg[s].wait()
                    ha.append(pltpu.async_copy(rows[s], acc_sh.at[cidx[s]],
                                               sema, add=True))
                issue_src(sidx, knext, semI)
                return ha

            def condg(k):
                return k + 16 * (2 * D - 1) < end

            def group(k):
                ha = half(sidxA, cidxA, rowsA, k, semia, k + STRIDE)
                hb = half(sidxB, cidxB, rowsB, k + 16 * D, semib,
                          k + STRIDE + 16 * D)
                for h in ha + hb:
                    h.wait()
                issue_comb(cidxA, k + STRIDE, semia)
                issue_comb(cidxB, k + STRIDE + 16 * D, semib)
                return k + STRIDE

            k = lax.while_loop(condg, group, k0)

            for (sidx, cidx, rows, kk, semI) in (
                    (sidxA, cidxA, rowsA, k, semia),
                    (sidxB, cidxB, rowsB, k + 16 * D, semib)):
                wait_idx(sidx, cidx, kk, semI)
                for s in range(D):
                    def _do(sidx=sidx, cidx=cidx, rows=rows, s=s):
                        pltpu.async_copy(h_hbm.at[sidx[s]], rows[s],
                                         semg).wait()
                        pltpu.async_copy(rows[s], acc_sh.at[cidx[s]],
                                         sema, add=True).wait()
                    pl.when(kk + 16 * s < end)(_do)

            plsc.subcore_barrier()

            def dcond(m):
                return m < NCK // 125

            def dbody(m):
                hd = [pltpu.async_copy(
                    acc_sh.at[pl.ds(r * PADC + m * 125, 125), :],
                    agg_hbm.at[r, pl.ds(b * NCK + m * 125, 125), :], semg)
                    for r in range(NUM_REL)]
                for h in hd:
                    h.wait()
                return m + 16

            lax.while_loop(dcond, dbody, tid)
            plsc.subcore_barrier()

    f = pl.kernel(
        body,
        out_type=jax.ShapeDtypeStruct((NUM_REL, n, 64), jnp.float32),
        mesh=_mesh(),
        scratch_types=[
            pltpu.VMEM((16,), jnp.int32),
        ] + [
            t
            for _ in range(D)
            for t in (pltpu.VMEM((128,), jnp.int32),
                      pltpu.VMEM((128,), jnp.int32),
                      pltpu.VMEM((128, 64), jnp.float32),
                      pltpu.VMEM((128,), jnp.int32),
                      pltpu.VMEM((128,), jnp.int32),
                      pltpu.VMEM((128, 64), jnp.float32))
        ] + [
            pltpu.VMEM_SHARED((NUM_REL * PADC, 64), jnp.float32),
            pltpu.SemaphoreType.DMA,
            pltpu.SemaphoreType.DMA,
            pltpu.SemaphoreType.DMA,
            pltpu.SemaphoreType.DMA,
        ],
        compiler_params=_sc_params(),
    )
    return f(h, src_part, comb_part, rowmeta, zeros_rows)


def _partition_glue(cnt_wb):
    cnt = cnt_wb[:, :NB]
    pc = (cnt + 7) & ~jnp.int32(7)
    u = jnp.sum(pc, axis=0)
    rows = (u + 255) // 128
    start_row = jnp.concatenate([jnp.zeros((1,), jnp.int32),
                                 jnp.cumsum(rows)]).astype(jnp.int32)
    start_ent = start_row[:NB] * 128
    slot_off = start_ent[None, :] + (jnp.cumsum(pc, axis=0) - pc)
    loc_off = jnp.cumsum(pc, axis=1) - pc
    dummy_a = start_ent + u
    dummy_b = start_row[1:] * 128 - 128

    def pad16(a):
        return jnp.concatenate(
            [a.astype(jnp.int32),
             jnp.zeros(a.shape[:-1] + (16 - a.shape[-1],), jnp.int32)], -1)

    rowmeta = pad16(start_row[None, :NB + 1])
    dummy_meta = jnp.stack([pad16(dummy_a[None, :])[0],
                            pad16(dummy_b[None, :])[0]])
    return pad16(slot_off), pad16(loc_off), rowmeta, dummy_meta


def kernel(x, edge_index, edge_type, batch, embed, W1, Wroot1, b1, W2, Wroot2, b2, linW, linb):
    n = x.shape[0]
    e = edge_index.shape[1]
    src, dst = edge_index[0], edge_index[1]

    npad = ((n + 128 * NW - 1) // (128 * NW)) * (128 * NW)
    x_pad = jnp.concatenate([x.astype(jnp.int32), jnp.zeros((npad - n,), jnp.int32)])
    h0 = _sc_embed_gather(x_pad, embed)

    epad = NW * ET
    src_p = jnp.concatenate([src.astype(jnp.int32), jnp.zeros((epad - e,), jnp.int32)])
    dst_p = jnp.concatenate([dst.astype(jnp.int32), jnp.full((epad - e,), n, jnp.int32)])
    typ_p = jnp.concatenate([edge_type.astype(jnp.int32), jnp.zeros((epad - e,), jnp.int32)])

    cnt_wb = _sc_count(dst_p)
    slot_off, loc_off, rowmeta, dummy_meta = _partition_glue(cnt_wb)
    src_part, comb_part = _sc_partition(src_p, dst_p, typ_p, slot_off, loc_off,
                                        cnt_wb, dummy_meta)
    hist = _sc_hist(comb_part, rowmeta)
    cnt = (hist.reshape(NB, 4, NUM_REL, PADC).sum(axis=1)[:, :, :NCK]
           .transpose(1, 0, 2).reshape(NUM_REL, n))
    icnt = (1.0 / jnp.maximum(cnt, 1.0)).T
    batch3d = batch.reshape(n // ROW_BLOCK, 1, ROW_BLOCK)
    zrows = jnp.zeros((128, 64), jnp.float32)

    agg1 = _sc_aggregate(h0, src_part, comb_part, rowmeta, zrows, n)
    h1 = _tc_dense(h0, agg1, icnt, Wroot1, b1, W1, n=n)
    agg2 = _sc_aggregate(h1, src_part, comb_part, rowmeta, zrows, n)
    h2 = _tc_dense(h1, agg2, icnt, Wroot2, b2, W2)
    return _tc_pool(h2, batch3d, linW, linb)
